# baseline (speedup 1.0000x reference)
import functools
import numpy as np
import jax
import jax.numpy as jnp
from jax import lax
from jax.experimental import pallas as pl
from jax.experimental.pallas import tpu as pltpu
from jax.experimental.pallas import tpu_sc as plsc

B, S, D, H, NMEL = 2, 8192, 768, 12, 80
DH = D // H
NB = 128
CH = 64
KS = 512
DFF = 3072
PRE = 256
BH = B * H
BHS = BH * S
ROWW = 4 * DH
BS = 512
NSB = S // BS
RC = 128
Nb_CS = S // RC
NW = 32
RPW = BHS // NW
TB = 128
NTB = RPW // TB


def _sinusoid_np():
    pos = np.arange(S)[:, None].astype(np.float64)
    i = np.arange(D)[None, :]
    ang = pos / np.power(10000.0, (2 * (i // 2)) / float(D))
    return np.where(i % 2 == 0, np.sin(ang), np.cos(ang)).astype(np.float32)


_PE_NP = _sinusoid_np()


def _ln(x, g, b):
    m = jnp.mean(x, -1, keepdims=True)
    v = jnp.mean((x - m) * (x - m), -1, keepdims=True)
    return (x - m) / jnp.sqrt(v + 1e-5) * g + b


def _prep_body(inp_ref, pe_ref, pW1_ref, pb1_ref, pW2_ref, pb2_ref,
               g1_ref, be1_ref, Wqk_ref, Wv_ref, rot_ref,
               h0_ref, tab_ref, bkt_ref):
    x = inp_ref[0]
    h = jnp.maximum(jnp.dot(x, pW1_ref[...]) + pb1_ref[...], 0.0)
    h = jnp.maximum(jnp.dot(h, pW2_ref[...]) + pb2_ref[...], 0.0)
    h = h + pe_ref[...]
    h0_ref[0] = h
    xn = _ln(h, g1_ref[...], be1_ref[...])
    qk = jnp.dot(xn, Wqk_ref[...])
    v = jnp.dot(xn, Wv_ref[...])
    sblk = pl.program_id(1)
    posf = (jnp.float32(sblk * BS)
            + lax.broadcasted_iota(jnp.int32, (BS, DH), 0).astype(jnp.float32))
    io64 = lax.broadcasted_iota(jnp.int32, (BS, DH), 1).astype(jnp.float32)
    bkts = []
    for hh in range(H):
        sl = slice(hh * DH, (hh + 1) * DH)
        qkh = qk[:, sl]
        tab_ref[0, hh, :, 0:DH] = qkh
        tab_ref[0, hh, :, DH:2 * DH] = v[:, sl]
        tab_ref[0, hh, :, 2 * DH:3 * DH] = posf
        r = jnp.dot(qkh, rot_ref[hh])
        m1 = jnp.max(r, -1, keepdims=True)
        i1 = jnp.min(jnp.where(r == m1, io64, 1e9), -1)
        m2 = jnp.max(-r, -1, keepdims=True)
        i2 = jnp.min(jnp.where(-r == m2, io64, 1e9), -1)
        b = jnp.where(m1[:, 0] >= m2[:, 0], i1, 64.0 + i2)
        bkts.append(b)
    bkt_ref[0] = jnp.stack(bkts, axis=0)


def _run_prep(input_, pe_scaled, pW1, pb1, pW2, pb2, g1, be1, Wqk, Wv, rot):
    full = lambda shp: pl.BlockSpec(shp, lambda b, s: (0,) * len(shp))
    return pl.pallas_call(
        _prep_body,
        grid=(B, NSB),
        in_specs=[
            pl.BlockSpec((1, BS, NMEL), lambda b, s: (b, s, 0)),
            pl.BlockSpec((BS, D), lambda b, s: (s, 0)),
            full((NMEL, PRE)), full((1, PRE)), full((PRE, D)), full((1, D)),
            full((1, D)), full((1, D)), full((D, D)), full((D, D)),
            full((H, DH, NB // 2)),
        ],
        out_specs=[
            pl.BlockSpec((1, BS, D), lambda b, s: (b, s, 0)),
            pl.BlockSpec((1, H, BS, ROWW), lambda b, s: (b, 0, s, 0)),
            pl.BlockSpec((1, H, BS), lambda b, s: (b, 0, s)),
        ],
        out_shape=[
            jax.ShapeDtypeStruct((B, S, D), jnp.float32),
            jax.ShapeDtypeStruct((B, H, S, ROWW), jnp.float32),
            jax.ShapeDtypeStruct((B, H, S), jnp.float32),
        ],
    )(input_, pe_scaled, pW1, pb1.reshape(1, PRE), pW2, pb2.reshape(1, D),
      g1.reshape(1, D), be1.reshape(1, D), Wqk, Wv, rot)


def _dest_body(bk_ref, out_ref):
    bk = bk_ref[0]
    iob = lax.broadcasted_iota(jnp.int32, (1, NB), 1).astype(jnp.float32)
    oh3 = (bk[:, :, None] == iob[None]).astype(jnp.float32)
    cnt = jnp.sum(oh3, axis=1)
    eq = (bk[:, :, None] == bk[:, None, :]).astype(jnp.float32)
    tri = (lax.broadcasted_iota(jnp.int32, (RC, RC), 1)
           < lax.broadcasted_iota(jnp.int32, (RC, RC), 0)).astype(jnp.float32)
    rank_local = jnp.sum(eq * tri[None], axis=2)
    ioc = lax.broadcasted_iota(jnp.int32, (Nb_CS, Nb_CS), 0)
    ioc2 = lax.broadcasted_iota(jnp.int32, (Nb_CS, Nb_CS), 1)
    ltri = (ioc2 < ioc).astype(jnp.float32)
    cp = jnp.sum(ltri[:, :, None] * cnt[None, :, :], axis=1)
    total = jnp.sum(cnt, axis=0, keepdims=True)
    totc = jnp.transpose(total)
    iom = lax.broadcasted_iota(jnp.int32, (NB, NB), 0)
    ion = lax.broadcasted_iota(jnp.int32, (NB, NB), 1)
    bstart = jnp.sum(jnp.where(iom < ion, totc, 0.0), axis=0,
                     keepdims=True)
    base = bstart + cp
    term1 = jnp.sum(oh3 * base[:, None, :], axis=2)
    dest = term1 + rank_local
    gdest = dest.astype(jnp.int32) + pl.program_id(0) * S
    out_ref[0] = gdest


def _run_dest(bkt_f):
    return pl.pallas_call(
        _dest_body,
        grid=(BH,),
        in_specs=[pl.BlockSpec((1, Nb_CS, RC), lambda i: (i, 0, 0))],
        out_specs=pl.BlockSpec((1, Nb_CS, RC), lambda i: (i, 0, 0)),
        out_shape=jax.ShapeDtypeStruct((BH, Nb_CS, RC), jnp.int32),
    )(bkt_f.reshape(BH, Nb_CS, RC))


@functools.lru_cache(maxsize=1)
def _get_sc_kernels():
    mesh = plsc.VectorSubcoreMesh(core_axis_name="c", subcore_axis_name="s")

    @functools.partial(
        pl.kernel, mesh=mesh,
        out_type=jax.ShapeDtypeStruct((BHS, ROWW), jnp.float32),
        scratch_types=[
            pltpu.VMEM((TB,), jnp.int32),
            pltpu.VMEM((TB, ROWW), jnp.float32),
            pltpu.SemaphoreType.DMA,
        ],
    )
    def sc_scatter(src_hbm, idx_hbm, out_hbm, idx_v, buf, sem):
        wid = lax.axis_index("s") * 2 + lax.axis_index("c")
        base = wid * RPW

        def body(j, carry):
            pltpu.sync_copy(idx_hbm.at[pl.ds(base + j * TB, TB)], idx_v)
            pltpu.sync_copy(src_hbm.at[pl.ds(base + j * TB, TB)], buf)
            pltpu.async_copy(buf, out_hbm.at[idx_v], sem).wait()
            return carry

        lax.fori_loop(0, NTB, body, 0)

    @functools.partial(
        pl.kernel, mesh=mesh,
        out_type=jax.ShapeDtypeStruct((BHS, 2 * DH), jnp.float32),
        scratch_types=[
            pltpu.VMEM((TB,), jnp.int32),
            pltpu.VMEM((TB, 2 * DH), jnp.float32),
            pltpu.SemaphoreType.DMA,
        ],
    )
    def sc_gather(src_hbm, idx_hbm, out_hbm, idx_v, buf, sem):
        wid = lax.axis_index("s") * 2 + lax.axis_index("c")
        base = wid * RPW

        def body(j, carry):
            pltpu.sync_copy(idx_hbm.at[pl.ds(base + j * TB, TB)], idx_v)
            pltpu.async_copy(src_hbm.at[idx_v], buf, sem).wait()
            pltpu.sync_copy(buf, out_hbm.at[pl.ds(base + j * TB, TB)])
            return carry

        lax.fori_loop(0, NTB, body, 0)

    return sc_scatter, sc_gather


RB = 512
CPB = RB // CH
NBANDS = S // RB


def _attn_body(main_ref, prev_ref, out_ref):
    blk = main_ref[0, 0]
    prv = prev_ref[0, 0]
    q = blk[:, 0:DH]
    kcat = jnp.concatenate([prv[:, 0:DH], q], axis=0)
    vcat = jnp.concatenate([prv[:, DH:2 * DH], blk[:, DH:2 * DH]], axis=0)
    pcat = jnp.concatenate([prv[:, 2 * DH:3 * DH], blk[:, 2 * DH:3 * DH]],
                           axis=0)
    nrm = jnp.sqrt(jnp.sum(kcat * kcat, -1, keepdims=True))
    kn = kcat / (nrm + 1e-6)
    qs = q * 0.125
    kpT = jnp.transpose(pcat[:, 0:1])
    qp = blk[:, 2 * DH:2 * DH + 1]
    for c in range(CPB):
        r0 = c * CH
        qc = qs[r0:r0 + CH]
        kc = kn[r0:r0 + 2 * CH]
        dc = lax.dot_general(qc, kc, (((1,), (1,)), ((), ())))
        kp_c = kpT[:, r0:r0 + 2 * CH]
        qp_c = qp[r0:r0 + CH]
        dc = jnp.where(kp_c > qp_c, -1e9, dc)
        m = jnp.max(dc, -1, keepdims=True)
        e = jnp.exp(dc - m)
        attn = e / jnp.sum(e, -1, keepdims=True)
        o = jnp.dot(attn, vcat[r0:r0 + 2 * CH])
        out_ref[0, 0, r0:r0 + CH, :] = jnp.concatenate([o, o], axis=1)


def _run_attn(sorted_tbl):
    return pl.pallas_call(
        _attn_body,
        grid=(B, H, NBANDS),
        in_specs=[
            pl.BlockSpec((1, 1, RB, ROWW), lambda b, h, i: (b, h, i, 0)),
            pl.BlockSpec((1, 1, CH, ROWW),
                         lambda b, h, i: (b, h, (i * CPB - 1) % (S // CH), 0)),
        ],
        out_specs=pl.BlockSpec((1, 1, RB, 2 * DH),
                               lambda b, h, i: (b, h, i, 0)),
        out_shape=jax.ShapeDtypeStruct((B, H, S, 2 * DH), jnp.float32),
    )(sorted_tbl, sorted_tbl)


def _kv_body(keys_ref, Wk_ref, Wv2_ref, k2_ref, v2_ref):
    k = keys_ref[0]
    k2_ref[0] = jnp.dot(k, Wk_ref[...])
    v2_ref[0] = jnp.dot(k, Wv2_ref[...])


def _run_kv(keys, Wk, Wv2):
    return pl.pallas_call(
        _kv_body,
        grid=(B,),
        in_specs=[
            pl.BlockSpec((1, KS, D), lambda b: (b, 0, 0)),
            pl.BlockSpec((D, D), lambda b: (0, 0)),
            pl.BlockSpec((D, D), lambda b: (0, 0)),
        ],
        out_specs=[
            pl.BlockSpec((1, KS, D), lambda b: (b, 0, 0)),
            pl.BlockSpec((1, KS, D), lambda b: (b, 0, 0)),
        ],
        out_shape=[
            jax.ShapeDtypeStruct((B, KS, D), jnp.float32),
            jax.ShapeDtypeStruct((B, KS, D), jnp.float32),
        ],
    )(keys, Wk, Wv2)


def _tail_body(h0_ref, ou_ref, Wo_ref, g2_ref, be2_ref, Wq_ref,
               k2_ref, v2_ref, Wo2_ref, g3_ref, be3_ref,
               Wf1_ref, bf1_ref, Wf2_ref, bf2_ref,
               Wm_ref, bm_ref, ws_ref, bs_ref,
               mel_ref, stop_ref):
    acc = jnp.zeros((BS, D), jnp.float32)
    for hh in range(H):
        sl = slice(hh * DH, (hh + 1) * DH)
        acc = acc + jnp.dot(ou_ref[0, hh][:, 0:DH], Wo_ref[sl, :])
    h1 = h0_ref[0] + acc
    xn2 = _ln(h1, g2_ref[...], be2_ref[...])
    cross = jnp.zeros((BS, D), jnp.float32)
    for hh in range(H):
        sl = slice(hh * DH, (hh + 1) * DH)
        qh = jnp.dot(xn2, Wq_ref[:, sl])
        kh = k2_ref[0][:, sl]
        dh = lax.dot_general(qh, kh, (((1,), (1,)), ((), ()))) / 8.0
        m = jnp.max(dh, -1, keepdims=True)
        e = jnp.exp(dh - m)
        ah = e / jnp.sum(e, -1, keepdims=True)
        oh = jnp.dot(ah, v2_ref[0][:, sl])
        cross = cross + jnp.dot(oh, Wo2_ref[sl, :])
    h2 = h1 + cross
    xn3 = _ln(h2, g3_ref[...], be3_ref[...])
    f = jnp.maximum(jnp.dot(xn3, Wf1_ref[...]) + bf1_ref[...], 0.0)
    h3 = h2 + jnp.dot(f, Wf2_ref[...]) + bf2_ref[...]
    mel_ref[0] = jnp.dot(h3, Wm_ref[...]) + bm_ref[...]
    stop_ref[0] = (jnp.sum(h3 * ws_ref[...], -1, keepdims=True)
                   + bs_ref[0, 0])


def _run_tail(h0, o_u, Wo, g2, be2, Wq, k2, v2, Wo2, g3, be3,
              Wf1, bf1, Wf2, bf2, Wm, bm, Ws, bs):
    full = lambda shp: pl.BlockSpec(shp, lambda b, s: (0,) * len(shp))
    return pl.pallas_call(
        _tail_body,
        grid=(B, NSB),
        in_specs=[
            pl.BlockSpec((1, BS, D), lambda b, s: (b, s, 0)),
            pl.BlockSpec((1, H, BS, 2 * DH), lambda b, s: (b, 0, s, 0)),
            full((D, D)), full((1, D)), full((1, D)), full((D, D)),
            pl.BlockSpec((1, KS, D), lambda b, s: (b, 0, 0)),
            pl.BlockSpec((1, KS, D), lambda b, s: (b, 0, 0)),
            full((D, D)), full((1, D)), full((1, D)),
            full((D, DFF)), full((1, DFF)), full((DFF, D)), full((1, D)),
            full((D, NMEL)), full((1, NMEL)), full((1, D)), full((1, 1)),
        ],
        out_specs=[
            pl.BlockSpec((1, BS, NMEL), lambda b, s: (b, s, 0)),
            pl.BlockSpec((1, BS, 1), lambda b, s: (b, s, 0)),
        ],
        out_shape=[
            jax.ShapeDtypeStruct((B, S, NMEL), jnp.float32),
            jax.ShapeDtypeStruct((B, S, 1), jnp.float32),
        ],
    )(h0, o_u, Wo, g2.reshape(1, D), be2.reshape(1, D), Wq, k2, v2, Wo2,
      g3.reshape(1, D), be3.reshape(1, D), Wf1, bf1.reshape(1, DFF), Wf2,
      bf2.reshape(1, D), Wm, bm.reshape(1, NMEL), Ws.reshape(1, D),
      bs.reshape(1, 1))


@jax.jit
def kernel(input_, keys, pW1, pb1, pW2, pb2, alpha, rot, g1, be1, Wqk, Wv,
           Wo, g2, be2, Wq, Wk, Wv2, Wo2, g3, be3, Wf1, bf1, Wf2, bf2,
           Wm, bm, Ws, bs):
    pe_scaled = alpha * jnp.asarray(_PE_NP)
    h0, table, bkt_f = _run_prep(input_, pe_scaled, pW1, pb1, pW2, pb2,
                                 g1, be1, Wqk, Wv, rot)
    gdest = _run_dest(bkt_f)
    gidx = gdest.reshape(BHS)
    sc_scatter, sc_gather = _get_sc_kernels()
    sorted_tbl = sc_scatter(table.reshape(BHS, ROWW), gidx)
    o_sorted = _run_attn(sorted_tbl.reshape(B, H, S, ROWW))
    o_u = sc_gather(o_sorted.reshape(BHS, 2 * DH), gidx).reshape(
        B, H, S, 2 * DH)
    k2, v2 = _run_kv(keys, Wk, Wv2)
    mel, stop = _run_tail(h0, o_u, Wo, g2, be2, Wq, k2, v2, Wo2, g3, be3,
                          Wf1, bf1, Wf2, bf2, Wm, bm, Ws, bs)
    return (mel, stop)


# speedup vs baseline: 1.3366x; 1.3366x over previous
import functools
import numpy as np
import jax
import jax.numpy as jnp
from jax import lax
from jax.experimental import pallas as pl
from jax.experimental.pallas import tpu as pltpu
from jax.experimental.pallas import tpu_sc as plsc

B, S, D, H, NMEL = 2, 8192, 768, 12, 80
DH = D // H
NB = 128
CH = 64
KS = 512
DFF = 3072
PRE = 256
BH = B * H
BHS = BH * S
ROWW = 4 * DH
BS = 512
NSB = S // BS
RC = 128
Nb_CS = S // RC
NW = 32
RPW = BHS // NW
TB = 128
NTB = RPW // TB


def _sinusoid_np():
    pos = np.arange(S)[:, None].astype(np.float64)
    i = np.arange(D)[None, :]
    ang = pos / np.power(10000.0, (2 * (i // 2)) / float(D))
    return np.where(i % 2 == 0, np.sin(ang), np.cos(ang)).astype(np.float32)


_PE_NP = _sinusoid_np()


def _ln(x, g, b):
    m = jnp.mean(x, -1, keepdims=True)
    v = jnp.mean((x - m) * (x - m), -1, keepdims=True)
    return (x - m) / jnp.sqrt(v + 1e-5) * g + b


def _prep_body(inp_ref, pe_ref, pW1_ref, pb1_ref, pW2_ref, pb2_ref,
               g1_ref, be1_ref, Wqk_ref, Wv_ref, rot_ref,
               h0_ref, tab_ref, bkt_ref):
    x = inp_ref[0]
    h = jnp.maximum(jnp.dot(x, pW1_ref[...]) + pb1_ref[...], 0.0)
    h = jnp.maximum(jnp.dot(h, pW2_ref[...]) + pb2_ref[...], 0.0)
    h = h + pe_ref[...]
    h0_ref[0] = h
    xn = _ln(h, g1_ref[...], be1_ref[...])
    qk = jnp.dot(xn, Wqk_ref[...])
    v = jnp.dot(xn, Wv_ref[...])
    sblk = pl.program_id(1)
    posf = (jnp.float32(sblk * BS)
            + lax.broadcasted_iota(jnp.int32, (BS, DH), 0).astype(jnp.float32))
    io64 = lax.broadcasted_iota(jnp.int32, (BS, DH), 1).astype(jnp.float32)
    bkts = []
    for hh in range(H):
        sl = slice(hh * DH, (hh + 1) * DH)
        qkh = qk[:, sl]
        tab_ref[0, hh, :, 0:DH] = qkh
        tab_ref[0, hh, :, DH:2 * DH] = v[:, sl]
        tab_ref[0, hh, :, 2 * DH:3 * DH] = posf
        r = jnp.dot(qkh, rot_ref[hh])
        m1 = jnp.max(r, -1, keepdims=True)
        i1 = jnp.min(jnp.where(r == m1, io64, 1e9), -1)
        m2 = jnp.max(-r, -1, keepdims=True)
        i2 = jnp.min(jnp.where(-r == m2, io64, 1e9), -1)
        b = jnp.where(m1[:, 0] >= m2[:, 0], i1, 64.0 + i2)
        bkts.append(b)
    bkt_ref[0] = jnp.stack(bkts, axis=0)


def _run_prep(input_, pe_scaled, pW1, pb1, pW2, pb2, g1, be1, Wqk, Wv, rot):
    full = lambda shp: pl.BlockSpec(shp, lambda b, s: (0,) * len(shp))
    return pl.pallas_call(
        _prep_body,
        grid=(B, NSB),
        in_specs=[
            pl.BlockSpec((1, BS, NMEL), lambda b, s: (b, s, 0)),
            pl.BlockSpec((BS, D), lambda b, s: (s, 0)),
            full((NMEL, PRE)), full((1, PRE)), full((PRE, D)), full((1, D)),
            full((1, D)), full((1, D)), full((D, D)), full((D, D)),
            full((H, DH, NB // 2)),
        ],
        out_specs=[
            pl.BlockSpec((1, BS, D), lambda b, s: (b, s, 0)),
            pl.BlockSpec((1, H, BS, ROWW), lambda b, s: (b, 0, s, 0)),
            pl.BlockSpec((1, H, BS), lambda b, s: (b, 0, s)),
        ],
        out_shape=[
            jax.ShapeDtypeStruct((B, S, D), jnp.float32),
            jax.ShapeDtypeStruct((B, H, S, ROWW), jnp.float32),
            jax.ShapeDtypeStruct((B, H, S), jnp.float32),
        ],
    )(input_, pe_scaled, pW1, pb1.reshape(1, PRE), pW2, pb2.reshape(1, D),
      g1.reshape(1, D), be1.reshape(1, D), Wqk, Wv, rot)


def _dest_body(bk_ref, out_ref):
    bk = bk_ref[0]
    iob = lax.broadcasted_iota(jnp.int32, (1, NB), 1).astype(jnp.float32)
    oh3 = (bk[:, :, None] == iob[None]).astype(jnp.float32)
    cnt = jnp.sum(oh3, axis=1)
    eq = (bk[:, :, None] == bk[:, None, :]).astype(jnp.float32)
    tri = (lax.broadcasted_iota(jnp.int32, (RC, RC), 1)
           < lax.broadcasted_iota(jnp.int32, (RC, RC), 0)).astype(jnp.float32)
    rank_local = jnp.sum(eq * tri[None], axis=2)
    ioc = lax.broadcasted_iota(jnp.int32, (Nb_CS, Nb_CS), 0)
    ioc2 = lax.broadcasted_iota(jnp.int32, (Nb_CS, Nb_CS), 1)
    ltri = (ioc2 < ioc).astype(jnp.float32)
    cp = jnp.sum(ltri[:, :, None] * cnt[None, :, :], axis=1)
    total = jnp.sum(cnt, axis=0, keepdims=True)
    totc = jnp.transpose(total)
    iom = lax.broadcasted_iota(jnp.int32, (NB, NB), 0)
    ion = lax.broadcasted_iota(jnp.int32, (NB, NB), 1)
    bstart = jnp.sum(jnp.where(iom < ion, totc, 0.0), axis=0,
                     keepdims=True)
    base = bstart + cp
    term1 = jnp.sum(oh3 * base[:, None, :], axis=2)
    dest = term1 + rank_local
    gdest = dest.astype(jnp.int32) + pl.program_id(0) * S
    out_ref[0] = gdest


def _run_dest(bkt_f):
    return pl.pallas_call(
        _dest_body,
        grid=(BH,),
        in_specs=[pl.BlockSpec((1, Nb_CS, RC), lambda i: (i, 0, 0))],
        out_specs=pl.BlockSpec((1, Nb_CS, RC), lambda i: (i, 0, 0)),
        out_shape=jax.ShapeDtypeStruct((BH, Nb_CS, RC), jnp.int32),
    )(bkt_f.reshape(BH, Nb_CS, RC))


@functools.lru_cache(maxsize=1)
def _get_sc_kernels():
    mesh = plsc.VectorSubcoreMesh(core_axis_name="c", subcore_axis_name="s")

    @functools.partial(
        pl.kernel, mesh=mesh,
        out_type=jax.ShapeDtypeStruct((BHS, ROWW), jnp.float32),
        scratch_types=[
            pltpu.VMEM((TB,), jnp.int32),
            pltpu.VMEM((TB, ROWW), jnp.float32),
            pltpu.SemaphoreType.DMA,
        ],
    )
    def sc_scatter(src_hbm, idx_hbm, out_hbm, idx_v, buf, sem):
        wid = lax.axis_index("s") * 2 + lax.axis_index("c")
        base = wid * RPW

        def body(j, carry):
            pltpu.sync_copy(idx_hbm.at[pl.ds(base + j * TB, TB)], idx_v)
            pltpu.sync_copy(src_hbm.at[pl.ds(base + j * TB, TB)], buf)
            pltpu.async_copy(buf, out_hbm.at[idx_v], sem).wait()
            return carry

        lax.fori_loop(0, NTB, body, 0)

    @functools.partial(
        pl.kernel, mesh=mesh,
        out_type=jax.ShapeDtypeStruct((BHS, 2 * DH), jnp.float32),
        scratch_types=[
            pltpu.VMEM((TB,), jnp.int32),
            pltpu.VMEM((TB, 2 * DH), jnp.float32),
            pltpu.SemaphoreType.DMA,
        ],
    )
    def sc_gather(src_hbm, idx_hbm, out_hbm, idx_v, buf, sem):
        wid = lax.axis_index("s") * 2 + lax.axis_index("c")
        base = wid * RPW

        def body(j, carry):
            pltpu.sync_copy(idx_hbm.at[pl.ds(base + j * TB, TB)], idx_v)
            pltpu.async_copy(src_hbm.at[idx_v], buf, sem).wait()
            pltpu.sync_copy(buf, out_hbm.at[pl.ds(base + j * TB, TB)])
            return carry

        lax.fori_loop(0, NTB, body, 0)

    return sc_scatter, sc_gather


RB = 512
CPB = RB // CH
NBANDS = S // RB


NBPG = 2


def _attn_body(main_ref, prev_ref, out_ref):
    blk = main_ref[0, 0]
    prv = prev_ref[0, 0]
    iq = lax.broadcasted_iota(jnp.int32, (RB, RB + CH), 0) // CH
    jk = lax.broadcasted_iota(jnp.int32, (RB, RB + CH), 1) // CH
    band_ok = (jk == iq) | (jk == iq + 1)
    bandadd = jnp.where(band_ok, 0.0, -1e9)

    def do_band(rows, prev_rows, base):
        q = rows[:, 0:DH]
        kcat = jnp.concatenate([prev_rows[:, 0:DH], q], axis=0)
        vcat = jnp.concatenate([prev_rows[:, DH:2 * DH], rows[:, DH:2 * DH]],
                               axis=0)
        pcat = jnp.concatenate([prev_rows[:, 2 * DH:2 * DH + 1],
                                rows[:, 2 * DH:2 * DH + 1]], axis=0)
        nrm = jnp.sqrt(jnp.sum(kcat * kcat, -1, keepdims=True))
        kn = kcat / (nrm + 1e-6)
        qs = q * 0.125
        dots = lax.dot_general(qs, kn, (((1,), (1,)), ((), ())))
        kp = jnp.transpose(pcat)
        qp = rows[:, 2 * DH:2 * DH + 1]
        dots = jnp.where(kp > qp, -1e9, dots) + bandadd
        m = jnp.max(dots, -1, keepdims=True)
        e = jnp.exp(dots - m)
        attn = e / jnp.sum(e, -1, keepdims=True)
        o = jnp.dot(attn, vcat)
        out_ref[0, 0, base:base + RB, :] = jnp.concatenate([o, o], axis=1)

    do_band(blk[0:RB], prv, 0)
    for bb in range(1, NBPG):
        do_band(blk[bb * RB:(bb + 1) * RB],
                blk[bb * RB - CH:bb * RB], bb * RB)


def _run_attn(sorted_tbl):
    return pl.pallas_call(
        _attn_body,
        grid=(B, H, NBANDS // NBPG),
        in_specs=[
            pl.BlockSpec((1, 1, NBPG * RB, ROWW),
                         lambda b, h, i: (b, h, i, 0)),
            pl.BlockSpec((1, 1, CH, ROWW),
                         lambda b, h, i: (b, h,
                                          (i * CPB * NBPG - 1) % (S // CH),
                                          0)),
        ],
        out_specs=pl.BlockSpec((1, 1, NBPG * RB, 2 * DH),
                               lambda b, h, i: (b, h, i, 0)),
        out_shape=jax.ShapeDtypeStruct((B, H, S, 2 * DH), jnp.float32),
    )(sorted_tbl, sorted_tbl)


def _kv_body(keys_ref, Wk_ref, Wv2_ref, k2_ref, v2_ref):
    k = keys_ref[0]
    k2_ref[0] = jnp.dot(k, Wk_ref[...])
    v2_ref[0] = jnp.dot(k, Wv2_ref[...])


def _run_kv(keys, Wk, Wv2):
    return pl.pallas_call(
        _kv_body,
        grid=(B,),
        in_specs=[
            pl.BlockSpec((1, KS, D), lambda b: (b, 0, 0)),
            pl.BlockSpec((D, D), lambda b: (0, 0)),
            pl.BlockSpec((D, D), lambda b: (0, 0)),
        ],
        out_specs=[
            pl.BlockSpec((1, KS, D), lambda b: (b, 0, 0)),
            pl.BlockSpec((1, KS, D), lambda b: (b, 0, 0)),
        ],
        out_shape=[
            jax.ShapeDtypeStruct((B, KS, D), jnp.float32),
            jax.ShapeDtypeStruct((B, KS, D), jnp.float32),
        ],
    )(keys, Wk, Wv2)


def _tail_body(h0_ref, ou_ref, Wo_ref, g2_ref, be2_ref, Wq_ref,
               k2_ref, v2_ref, Wo2_ref, g3_ref, be3_ref,
               Wf1_ref, bf1_ref, Wf2_ref, bf2_ref,
               Wm_ref, bm_ref, ws_ref, bs_ref,
               mel_ref, stop_ref):
    acc = jnp.zeros((BS, D), jnp.float32)
    for hh in range(H):
        sl = slice(hh * DH, (hh + 1) * DH)
        acc = acc + jnp.dot(ou_ref[0, hh][:, 0:DH], Wo_ref[sl, :])
    h1 = h0_ref[0] + acc
    xn2 = _ln(h1, g2_ref[...], be2_ref[...])
    cross = jnp.zeros((BS, D), jnp.float32)
    for hh in range(H):
        sl = slice(hh * DH, (hh + 1) * DH)
        qh = jnp.dot(xn2, Wq_ref[:, sl])
        kh = k2_ref[0][:, sl]
        dh = lax.dot_general(qh, kh, (((1,), (1,)), ((), ()))) / 8.0
        m = jnp.max(dh, -1, keepdims=True)
        e = jnp.exp(dh - m)
        ah = e / jnp.sum(e, -1, keepdims=True)
        oh = jnp.dot(ah, v2_ref[0][:, sl])
        cross = cross + jnp.dot(oh, Wo2_ref[sl, :])
    h2 = h1 + cross
    xn3 = _ln(h2, g3_ref[...], be3_ref[...])
    f = jnp.maximum(jnp.dot(xn3, Wf1_ref[...]) + bf1_ref[...], 0.0)
    h3 = h2 + jnp.dot(f, Wf2_ref[...]) + bf2_ref[...]
    mel_ref[0] = jnp.dot(h3, Wm_ref[...]) + bm_ref[...]
    stop_ref[0] = (jnp.sum(h3 * ws_ref[...], -1, keepdims=True)
                   + bs_ref[0, 0])


def _run_tail(h0, o_u, Wo, g2, be2, Wq, k2, v2, Wo2, g3, be3,
              Wf1, bf1, Wf2, bf2, Wm, bm, Ws, bs):
    full = lambda shp: pl.BlockSpec(shp, lambda b, s: (0,) * len(shp))
    return pl.pallas_call(
        _tail_body,
        grid=(B, NSB),
        in_specs=[
            pl.BlockSpec((1, BS, D), lambda b, s: (b, s, 0)),
            pl.BlockSpec((1, H, BS, 2 * DH), lambda b, s: (b, 0, s, 0)),
            full((D, D)), full((1, D)), full((1, D)), full((D, D)),
            pl.BlockSpec((1, KS, D), lambda b, s: (b, 0, 0)),
            pl.BlockSpec((1, KS, D), lambda b, s: (b, 0, 0)),
            full((D, D)), full((1, D)), full((1, D)),
            full((D, DFF)), full((1, DFF)), full((DFF, D)), full((1, D)),
            full((D, NMEL)), full((1, NMEL)), full((1, D)), full((1, 1)),
        ],
        out_specs=[
            pl.BlockSpec((1, BS, NMEL), lambda b, s: (b, s, 0)),
            pl.BlockSpec((1, BS, 1), lambda b, s: (b, s, 0)),
        ],
        out_shape=[
            jax.ShapeDtypeStruct((B, S, NMEL), jnp.float32),
            jax.ShapeDtypeStruct((B, S, 1), jnp.float32),
        ],
    )(h0, o_u, Wo, g2.reshape(1, D), be2.reshape(1, D), Wq, k2, v2, Wo2,
      g3.reshape(1, D), be3.reshape(1, D), Wf1, bf1.reshape(1, DFF), Wf2,
      bf2.reshape(1, D), Wm, bm.reshape(1, NMEL), Ws.reshape(1, D),
      bs.reshape(1, 1))


@jax.jit
def kernel(input_, keys, pW1, pb1, pW2, pb2, alpha, rot, g1, be1, Wqk, Wv,
           Wo, g2, be2, Wq, Wk, Wv2, Wo2, g3, be3, Wf1, bf1, Wf2, bf2,
           Wm, bm, Ws, bs):
    pe_scaled = alpha * jnp.asarray(_PE_NP)
    h0, table, bkt_f = _run_prep(input_, pe_scaled, pW1, pb1, pW2, pb2,
                                 g1, be1, Wqk, Wv, rot)
    gdest = _run_dest(bkt_f)
    gidx = gdest.reshape(BHS)
    sc_scatter, sc_gather = _get_sc_kernels()
    sorted_tbl = sc_scatter(table.reshape(BHS, ROWW), gidx)
    o_sorted = _run_attn(sorted_tbl.reshape(B, H, S, ROWW))
    o_u = sc_gather(o_sorted.reshape(BHS, 2 * DH), gidx).reshape(
        B, H, S, 2 * DH)
    k2, v2 = _run_kv(keys, Wk, Wv2)
    mel, stop = _run_tail(h0, o_u, Wo, g2, be2, Wq, k2, v2, Wo2, g3, be3,
                          Wf1, bf1, Wf2, bf2, Wm, bm, Ws, bs)
    return (mel, stop)


# speedup vs baseline: 1.4182x; 1.0611x over previous
import functools
import numpy as np
import jax
import jax.numpy as jnp
from jax import lax
from jax.experimental import pallas as pl
from jax.experimental.pallas import tpu as pltpu
from jax.experimental.pallas import tpu_sc as plsc

B, S, D, H, NMEL = 2, 8192, 768, 12, 80
DH = D // H
NB = 128
CH = 64
KS = 512
DFF = 3072
PRE = 256
BH = B * H
BHS = BH * S
ROWW = 4 * DH
BS = 512
NSB = S // BS
RC = 128
Nb_CS = S // RC
NW = 32
RPW = BHS // NW
TB = 128
NTB = RPW // TB


def _sinusoid_np():
    pos = np.arange(S)[:, None].astype(np.float64)
    i = np.arange(D)[None, :]
    ang = pos / np.power(10000.0, (2 * (i // 2)) / float(D))
    return np.where(i % 2 == 0, np.sin(ang), np.cos(ang)).astype(np.float32)


_PE_NP = _sinusoid_np()


def _ln(x, g, b):
    m = jnp.mean(x, -1, keepdims=True)
    v = jnp.mean((x - m) * (x - m), -1, keepdims=True)
    return (x - m) / jnp.sqrt(v + 1e-5) * g + b


def _prep_body(inp_ref, pe_ref, pW1_ref, pb1_ref, pW2_ref, pb2_ref,
               g1_ref, be1_ref, Wqk_ref, Wv_ref, rot_ref,
               h0_ref, tab_ref, bkt_ref):
    x = inp_ref[0]
    h = jnp.maximum(jnp.dot(x, pW1_ref[...]) + pb1_ref[...], 0.0)
    h = jnp.maximum(jnp.dot(h, pW2_ref[...]) + pb2_ref[...], 0.0)
    h = h + pe_ref[...]
    h0_ref[0] = h
    xn = _ln(h, g1_ref[...], be1_ref[...])
    qk = jnp.dot(xn, Wqk_ref[...])
    v = jnp.dot(xn, Wv_ref[...])
    sblk = pl.program_id(1)
    posf = (jnp.float32(sblk * BS)
            + lax.broadcasted_iota(jnp.int32, (BS, DH), 0).astype(jnp.float32))
    io64 = lax.broadcasted_iota(jnp.int32, (BS, DH), 1).astype(jnp.float32)
    bkts = []
    for hh in range(H):
        sl = slice(hh * DH, (hh + 1) * DH)
        qkh = qk[:, sl]
        tab_ref[0, hh, :, 0:DH] = qkh
        tab_ref[0, hh, :, DH:2 * DH] = v[:, sl]
        tab_ref[0, hh, :, 2 * DH:3 * DH] = posf
        r = jnp.dot(qkh, rot_ref[hh])
        m1 = jnp.max(r, -1, keepdims=True)
        i1 = jnp.min(jnp.where(r == m1, io64, 1e9), -1)
        m2 = jnp.max(-r, -1, keepdims=True)
        i2 = jnp.min(jnp.where(-r == m2, io64, 1e9), -1)
        b = jnp.where(m1[:, 0] >= m2[:, 0], i1, 64.0 + i2)
        bkts.append(b)
    bkt_ref[0] = jnp.stack(bkts, axis=0)


def _run_prep(input_, pe_scaled, pW1, pb1, pW2, pb2, g1, be1, Wqk, Wv, rot):
    full = lambda shp: pl.BlockSpec(shp, lambda b, s: (0,) * len(shp))
    return pl.pallas_call(
        _prep_body,
        grid=(B, NSB),
        in_specs=[
            pl.BlockSpec((1, BS, NMEL), lambda b, s: (b, s, 0)),
            pl.BlockSpec((BS, D), lambda b, s: (s, 0)),
            full((NMEL, PRE)), full((1, PRE)), full((PRE, D)), full((1, D)),
            full((1, D)), full((1, D)), full((D, D)), full((D, D)),
            full((H, DH, NB // 2)),
        ],
        out_specs=[
            pl.BlockSpec((1, BS, D), lambda b, s: (b, s, 0)),
            pl.BlockSpec((1, H, BS, ROWW), lambda b, s: (b, 0, s, 0)),
            pl.BlockSpec((1, H, BS), lambda b, s: (b, 0, s)),
        ],
        out_shape=[
            jax.ShapeDtypeStruct((B, S, D), jnp.float32),
            jax.ShapeDtypeStruct((B, H, S, ROWW), jnp.float32),
            jax.ShapeDtypeStruct((B, H, S), jnp.float32),
        ],
    )(input_, pe_scaled, pW1, pb1.reshape(1, PRE), pW2, pb2.reshape(1, D),
      g1.reshape(1, D), be1.reshape(1, D), Wqk, Wv, rot)


def _dest_body(bk_ref, out_ref):
    bk = bk_ref[0]
    iob = lax.broadcasted_iota(jnp.int32, (1, NB), 1).astype(jnp.float32)
    oh3 = (bk[:, :, None] == iob[None]).astype(jnp.float32)
    cnt = jnp.sum(oh3, axis=1)
    eq = (bk[:, :, None] == bk[:, None, :]).astype(jnp.float32)
    tri = (lax.broadcasted_iota(jnp.int32, (RC, RC), 1)
           < lax.broadcasted_iota(jnp.int32, (RC, RC), 0)).astype(jnp.float32)
    rank_local = jnp.sum(eq * tri[None], axis=2)
    ioc = lax.broadcasted_iota(jnp.int32, (Nb_CS, Nb_CS), 0)
    ioc2 = lax.broadcasted_iota(jnp.int32, (Nb_CS, Nb_CS), 1)
    ltri = (ioc2 < ioc).astype(jnp.float32)
    cp = jnp.sum(ltri[:, :, None] * cnt[None, :, :], axis=1)
    total = jnp.sum(cnt, axis=0, keepdims=True)
    totc = jnp.transpose(total)
    iom = lax.broadcasted_iota(jnp.int32, (NB, NB), 0)
    ion = lax.broadcasted_iota(jnp.int32, (NB, NB), 1)
    bstart = jnp.sum(jnp.where(iom < ion, totc, 0.0), axis=0,
                     keepdims=True)
    base = bstart + cp
    term1 = jnp.sum(oh3 * base[:, None, :], axis=2)
    dest = term1 + rank_local
    gdest = dest.astype(jnp.int32) + pl.program_id(0) * S
    out_ref[0] = gdest


def _run_dest(bkt_f):
    return pl.pallas_call(
        _dest_body,
        grid=(BH,),
        in_specs=[pl.BlockSpec((1, Nb_CS, RC), lambda i: (i, 0, 0))],
        out_specs=pl.BlockSpec((1, Nb_CS, RC), lambda i: (i, 0, 0)),
        out_shape=jax.ShapeDtypeStruct((BH, Nb_CS, RC), jnp.int32),
    )(bkt_f.reshape(BH, Nb_CS, RC))


@functools.lru_cache(maxsize=1)
def _get_sc_kernels():
    mesh = plsc.VectorSubcoreMesh(core_axis_name="c", subcore_axis_name="s")

    @functools.partial(
        pl.kernel, mesh=mesh,
        out_type=jax.ShapeDtypeStruct((BHS, ROWW), jnp.float32),
        scratch_types=[
            pltpu.VMEM((TB,), jnp.int32),
            pltpu.VMEM((TB, ROWW), jnp.float32),
            pltpu.SemaphoreType.DMA,
        ],
    )
    def sc_scatter(src_hbm, idx_hbm, out_hbm, idx_v, buf, sem):
        wid = lax.axis_index("s") * 2 + lax.axis_index("c")
        base = wid * RPW

        def body(j, carry):
            pltpu.sync_copy(idx_hbm.at[pl.ds(base + j * TB, TB)], idx_v)
            pltpu.sync_copy(src_hbm.at[pl.ds(base + j * TB, TB)], buf)
            pltpu.async_copy(buf, out_hbm.at[idx_v], sem).wait()
            return carry

        lax.fori_loop(0, NTB, body, 0)

    @functools.partial(
        pl.kernel, mesh=mesh,
        out_type=jax.ShapeDtypeStruct((BHS, 2 * DH), jnp.float32),
        scratch_types=[
            pltpu.VMEM((TB,), jnp.int32),
            pltpu.VMEM((TB, 2 * DH), jnp.float32),
            pltpu.SemaphoreType.DMA,
        ],
    )
    def sc_gather(src_hbm, idx_hbm, out_hbm, idx_v, buf, sem):
        wid = lax.axis_index("s") * 2 + lax.axis_index("c")
        base = wid * RPW

        def body(j, carry):
            pltpu.sync_copy(idx_hbm.at[pl.ds(base + j * TB, TB)], idx_v)
            pltpu.async_copy(src_hbm.at[idx_v], buf, sem).wait()
            pltpu.sync_copy(buf, out_hbm.at[pl.ds(base + j * TB, TB)])
            return carry

        lax.fori_loop(0, NTB, body, 0)

    return sc_scatter, sc_gather


RB = 512
CPB = RB // CH
NBANDS = S // RB


NBPG = 4


def _attn_body(main_ref, prev_ref, out_ref):
    blk = main_ref[0, 0]
    prv = prev_ref[0, 0]
    iq = lax.broadcasted_iota(jnp.int32, (RB, RB + CH), 0) // CH
    jk = lax.broadcasted_iota(jnp.int32, (RB, RB + CH), 1) // CH
    band_ok = (jk == iq) | (jk == iq + 1)
    bandadd = jnp.where(band_ok, 0.0, -1e9)

    def do_band(rows, prev_rows, base):
        q = rows[:, 0:DH]
        kcat = jnp.concatenate([prev_rows[:, 0:DH], q], axis=0)
        vcat = jnp.concatenate([prev_rows[:, DH:2 * DH], rows[:, DH:2 * DH]],
                               axis=0)
        pcat = jnp.concatenate([prev_rows[:, 2 * DH:2 * DH + 1],
                                rows[:, 2 * DH:2 * DH + 1]], axis=0)
        nrm = jnp.sqrt(jnp.sum(kcat * kcat, -1, keepdims=True))
        kn = kcat / (nrm + 1e-6)
        qs = q * 0.125
        dots = lax.dot_general(qs, kn, (((1,), (1,)), ((), ())))
        kp = jnp.transpose(pcat)
        qp = rows[:, 2 * DH:2 * DH + 1]
        dots = jnp.where(kp > qp, -1e9, dots) + bandadd
        m = jnp.max(dots, -1, keepdims=True)
        e = jnp.exp(dots - m)
        attn = e / jnp.sum(e, -1, keepdims=True)
        o = jnp.dot(attn, vcat)
        out_ref[0, 0, base:base + RB, :] = jnp.concatenate([o, o], axis=1)

    do_band(blk[0:RB], prv, 0)
    for bb in range(1, NBPG):
        do_band(blk[bb * RB:(bb + 1) * RB],
                blk[bb * RB - CH:bb * RB], bb * RB)


def _run_attn(sorted_tbl):
    return pl.pallas_call(
        _attn_body,
        grid=(B, H, NBANDS // NBPG),
        in_specs=[
            pl.BlockSpec((1, 1, NBPG * RB, ROWW),
                         lambda b, h, i: (b, h, i, 0)),
            pl.BlockSpec((1, 1, CH, ROWW),
                         lambda b, h, i: (b, h,
                                          (i * CPB * NBPG - 1) % (S // CH),
                                          0)),
        ],
        out_specs=pl.BlockSpec((1, 1, NBPG * RB, 2 * DH),
                               lambda b, h, i: (b, h, i, 0)),
        out_shape=jax.ShapeDtypeStruct((B, H, S, 2 * DH), jnp.float32),
    )(sorted_tbl, sorted_tbl)


def _kv_body(keys_ref, Wk_ref, Wv2_ref, k2_ref, v2_ref):
    k = keys_ref[0]
    k2_ref[0] = jnp.dot(k, Wk_ref[...])
    v2_ref[0] = jnp.dot(k, Wv2_ref[...])


def _run_kv(keys, Wk, Wv2):
    return pl.pallas_call(
        _kv_body,
        grid=(B,),
        in_specs=[
            pl.BlockSpec((1, KS, D), lambda b: (b, 0, 0)),
            pl.BlockSpec((D, D), lambda b: (0, 0)),
            pl.BlockSpec((D, D), lambda b: (0, 0)),
        ],
        out_specs=[
            pl.BlockSpec((1, KS, D), lambda b: (b, 0, 0)),
            pl.BlockSpec((1, KS, D), lambda b: (b, 0, 0)),
        ],
        out_shape=[
            jax.ShapeDtypeStruct((B, KS, D), jnp.float32),
            jax.ShapeDtypeStruct((B, KS, D), jnp.float32),
        ],
    )(keys, Wk, Wv2)


def _tail_body(h0_ref, ou_ref, Wo_ref, g2_ref, be2_ref, Wq_ref,
               k2_ref, v2_ref, Wo2_ref, g3_ref, be3_ref,
               Wf1_ref, bf1_ref, Wf2_ref, bf2_ref,
               Wm_ref, bm_ref, ws_ref, bs_ref,
               mel_ref, stop_ref):
    acc = jnp.zeros((BS, D), jnp.float32)
    for hh in range(H):
        sl = slice(hh * DH, (hh + 1) * DH)
        acc = acc + jnp.dot(ou_ref[0, hh][:, 0:DH], Wo_ref[sl, :])
    h1 = h0_ref[0] + acc
    xn2 = _ln(h1, g2_ref[...], be2_ref[...])
    cross = jnp.zeros((BS, D), jnp.float32)
    for hh in range(H):
        sl = slice(hh * DH, (hh + 1) * DH)
        qh = jnp.dot(xn2, Wq_ref[:, sl])
        kh = k2_ref[0][:, sl]
        dh = lax.dot_general(qh, kh, (((1,), (1,)), ((), ()))) / 8.0
        m = jnp.max(dh, -1, keepdims=True)
        e = jnp.exp(dh - m)
        ah = e / jnp.sum(e, -1, keepdims=True)
        oh = jnp.dot(ah, v2_ref[0][:, sl])
        cross = cross + jnp.dot(oh, Wo2_ref[sl, :])
    h2 = h1 + cross
    xn3 = _ln(h2, g3_ref[...], be3_ref[...])
    f = jnp.maximum(jnp.dot(xn3, Wf1_ref[...]) + bf1_ref[...], 0.0)
    h3 = h2 + jnp.dot(f, Wf2_ref[...]) + bf2_ref[...]
    mel_ref[0] = jnp.dot(h3, Wm_ref[...]) + bm_ref[...]
    stop_ref[0] = (jnp.sum(h3 * ws_ref[...], -1, keepdims=True)
                   + bs_ref[0, 0])


def _run_tail(h0, o_u, Wo, g2, be2, Wq, k2, v2, Wo2, g3, be3,
              Wf1, bf1, Wf2, bf2, Wm, bm, Ws, bs):
    full = lambda shp: pl.BlockSpec(shp, lambda b, s: (0,) * len(shp))
    return pl.pallas_call(
        _tail_body,
        grid=(B, NSB),
        in_specs=[
            pl.BlockSpec((1, BS, D), lambda b, s: (b, s, 0)),
            pl.BlockSpec((1, H, BS, 2 * DH), lambda b, s: (b, 0, s, 0)),
            full((D, D)), full((1, D)), full((1, D)), full((D, D)),
            pl.BlockSpec((1, KS, D), lambda b, s: (b, 0, 0)),
            pl.BlockSpec((1, KS, D), lambda b, s: (b, 0, 0)),
            full((D, D)), full((1, D)), full((1, D)),
            full((D, DFF)), full((1, DFF)), full((DFF, D)), full((1, D)),
            full((D, NMEL)), full((1, NMEL)), full((1, D)), full((1, 1)),
        ],
        out_specs=[
            pl.BlockSpec((1, BS, NMEL), lambda b, s: (b, s, 0)),
            pl.BlockSpec((1, BS, 1), lambda b, s: (b, s, 0)),
        ],
        out_shape=[
            jax.ShapeDtypeStruct((B, S, NMEL), jnp.float32),
            jax.ShapeDtypeStruct((B, S, 1), jnp.float32),
        ],
    )(h0, o_u, Wo, g2.reshape(1, D), be2.reshape(1, D), Wq, k2, v2, Wo2,
      g3.reshape(1, D), be3.reshape(1, D), Wf1, bf1.reshape(1, DFF), Wf2,
      bf2.reshape(1, D), Wm, bm.reshape(1, NMEL), Ws.reshape(1, D),
      bs.reshape(1, 1))


@jax.jit
def kernel(input_, keys, pW1, pb1, pW2, pb2, alpha, rot, g1, be1, Wqk, Wv,
           Wo, g2, be2, Wq, Wk, Wv2, Wo2, g3, be3, Wf1, bf1, Wf2, bf2,
           Wm, bm, Ws, bs):
    pe_scaled = alpha * jnp.asarray(_PE_NP)
    h0, table, bkt_f = _run_prep(input_, pe_scaled, pW1, pb1, pW2, pb2,
                                 g1, be1, Wqk, Wv, rot)
    gdest = _run_dest(bkt_f)
    gidx = gdest.reshape(BHS)
    sc_scatter, sc_gather = _get_sc_kernels()
    sorted_tbl = sc_scatter(table.reshape(BHS, ROWW), gidx)
    o_sorted = _run_attn(sorted_tbl.reshape(B, H, S, ROWW))
    o_u = sc_gather(o_sorted.reshape(BHS, 2 * DH), gidx).reshape(
        B, H, S, 2 * DH)
    k2, v2 = _run_kv(keys, Wk, Wv2)
    mel, stop = _run_tail(h0, o_u, Wo, g2, be2, Wq, k2, v2, Wo2, g3, be3,
                          Wf1, bf1, Wf2, bf2, Wm, bm, Ws, bs)
    return (mel, stop)


# speedup vs baseline: 1.4761x; 1.0408x over previous
import functools
import numpy as np
import jax
import jax.numpy as jnp
from jax import lax
from jax.experimental import pallas as pl
from jax.experimental.pallas import tpu as pltpu
from jax.experimental.pallas import tpu_sc as plsc

B, S, D, H, NMEL = 2, 8192, 768, 12, 80
DH = D // H
NB = 128
CH = 64
KS = 512
DFF = 3072
PRE = 256
BH = B * H
BHS = BH * S
ROWW = 4 * DH
BS = 512
NSB = S // BS
RC = 128
Nb_CS = S // RC
NW = 32
RPW = BHS // NW
TB = 128
NTB = RPW // TB


def _sinusoid_np():
    pos = np.arange(S)[:, None].astype(np.float64)
    i = np.arange(D)[None, :]
    ang = pos / np.power(10000.0, (2 * (i // 2)) / float(D))
    return np.where(i % 2 == 0, np.sin(ang), np.cos(ang)).astype(np.float32)


_PE_NP = _sinusoid_np()


def _ln(x, g, b):
    m = jnp.mean(x, -1, keepdims=True)
    v = jnp.mean((x - m) * (x - m), -1, keepdims=True)
    return (x - m) / jnp.sqrt(v + 1e-5) * g + b


def _prep_body(inp_ref, pe_ref, pW1_ref, pb1_ref, pW2_ref, pb2_ref,
               g1_ref, be1_ref, Wqk_ref, Wv_ref, rot_ref,
               h0_ref, tab_ref, bkt_ref):
    x = inp_ref[0]
    h = jnp.maximum(jnp.dot(x, pW1_ref[...]) + pb1_ref[...], 0.0)
    h = jnp.maximum(jnp.dot(h, pW2_ref[...]) + pb2_ref[...], 0.0)
    h = h + pe_ref[...]
    h0_ref[0] = h
    xn = _ln(h, g1_ref[...], be1_ref[...])
    qk = jnp.dot(xn, Wqk_ref[...])
    v = jnp.dot(xn, Wv_ref[...])
    sblk = pl.program_id(1)
    posf = (jnp.float32(sblk * BS)
            + lax.broadcasted_iota(jnp.int32, (BS, DH), 0).astype(jnp.float32))
    io64 = lax.broadcasted_iota(jnp.int32, (BS, DH), 1).astype(jnp.float32)
    bkts = []
    for hh in range(H):
        sl = slice(hh * DH, (hh + 1) * DH)
        qkh = qk[:, sl]
        tab_ref[0, hh, :, 0:DH] = qkh
        tab_ref[0, hh, :, DH:2 * DH] = v[:, sl]
        tab_ref[0, hh, :, 2 * DH:3 * DH] = posf
        r = jnp.dot(qkh, rot_ref[hh])
        m1 = jnp.max(r, -1, keepdims=True)
        i1 = jnp.min(jnp.where(r == m1, io64, 1e9), -1)
        m2 = jnp.max(-r, -1, keepdims=True)
        i2 = jnp.min(jnp.where(-r == m2, io64, 1e9), -1)
        b = jnp.where(m1[:, 0] >= m2[:, 0], i1, 64.0 + i2)
        bkts.append(b)
    bkt_ref[0] = jnp.stack(bkts, axis=0)


def _run_prep(input_, pe_scaled, pW1, pb1, pW2, pb2, g1, be1, Wqk, Wv, rot):
    full = lambda shp: pl.BlockSpec(shp, lambda b, s: (0,) * len(shp))
    return pl.pallas_call(
        _prep_body,
        grid=(B, NSB),
        in_specs=[
            pl.BlockSpec((1, BS, NMEL), lambda b, s: (b, s, 0)),
            pl.BlockSpec((BS, D), lambda b, s: (s, 0)),
            full((NMEL, PRE)), full((1, PRE)), full((PRE, D)), full((1, D)),
            full((1, D)), full((1, D)), full((D, D)), full((D, D)),
            full((H, DH, NB // 2)),
        ],
        out_specs=[
            pl.BlockSpec((1, BS, D), lambda b, s: (b, s, 0)),
            pl.BlockSpec((1, H, BS, ROWW), lambda b, s: (b, 0, s, 0)),
            pl.BlockSpec((1, H, BS), lambda b, s: (b, 0, s)),
        ],
        out_shape=[
            jax.ShapeDtypeStruct((B, S, D), jnp.float32),
            jax.ShapeDtypeStruct((B, H, S, ROWW), jnp.float32),
            jax.ShapeDtypeStruct((B, H, S), jnp.float32),
        ],
    )(input_, pe_scaled, pW1, pb1.reshape(1, PRE), pW2, pb2.reshape(1, D),
      g1.reshape(1, D), be1.reshape(1, D), Wqk, Wv, rot)


def _dest_body(bk_ref, out_ref):
    bk = bk_ref[0]
    iob = lax.broadcasted_iota(jnp.int32, (1, NB), 1).astype(jnp.float32)
    oh3 = (bk[:, :, None] == iob[None]).astype(jnp.float32)
    cnt = jnp.sum(oh3, axis=1)
    eq = (bk[:, :, None] == bk[:, None, :]).astype(jnp.float32)
    tri = (lax.broadcasted_iota(jnp.int32, (RC, RC), 1)
           < lax.broadcasted_iota(jnp.int32, (RC, RC), 0)).astype(jnp.float32)
    rank_local = jnp.sum(eq * tri[None], axis=2)
    ioc = lax.broadcasted_iota(jnp.int32, (Nb_CS, Nb_CS), 0)
    ioc2 = lax.broadcasted_iota(jnp.int32, (Nb_CS, Nb_CS), 1)
    ltri = (ioc2 < ioc).astype(jnp.float32)
    cp = jnp.sum(ltri[:, :, None] * cnt[None, :, :], axis=1)
    total = jnp.sum(cnt, axis=0, keepdims=True)
    totc = jnp.transpose(total)
    iom = lax.broadcasted_iota(jnp.int32, (NB, NB), 0)
    ion = lax.broadcasted_iota(jnp.int32, (NB, NB), 1)
    bstart = jnp.sum(jnp.where(iom < ion, totc, 0.0), axis=0,
                     keepdims=True)
    base = bstart + cp
    term1 = jnp.sum(oh3 * base[:, None, :], axis=2)
    dest = term1 + rank_local
    gdest = dest.astype(jnp.int32) + pl.program_id(0) * S
    out_ref[0] = gdest


def _run_dest(bkt_f):
    return pl.pallas_call(
        _dest_body,
        grid=(BH,),
        in_specs=[pl.BlockSpec((1, Nb_CS, RC), lambda i: (i, 0, 0))],
        out_specs=pl.BlockSpec((1, Nb_CS, RC), lambda i: (i, 0, 0)),
        out_shape=jax.ShapeDtypeStruct((BH, Nb_CS, RC), jnp.int32),
    )(bkt_f.reshape(BH, Nb_CS, RC))


@functools.lru_cache(maxsize=1)
def _get_sc_kernels():
    mesh = plsc.VectorSubcoreMesh(core_axis_name="c", subcore_axis_name="s")

    @functools.partial(
        pl.kernel, mesh=mesh,
        out_type=jax.ShapeDtypeStruct((BHS, ROWW), jnp.float32),
        scratch_types=[
            pltpu.VMEM((TB,), jnp.int32),
            pltpu.VMEM((TB, ROWW), jnp.float32),
            pltpu.SemaphoreType.DMA,
        ],
    )
    def sc_scatter(src_hbm, idx_hbm, out_hbm, idx_v, buf, sem):
        wid = lax.axis_index("s") * 2 + lax.axis_index("c")
        base = wid * RPW

        def body(j, carry):
            pltpu.sync_copy(idx_hbm.at[pl.ds(base + j * TB, TB)], idx_v)
            pltpu.sync_copy(src_hbm.at[pl.ds(base + j * TB, TB)], buf)
            pltpu.async_copy(buf, out_hbm.at[idx_v], sem).wait()
            return carry

        lax.fori_loop(0, NTB, body, 0)

    @functools.partial(
        pl.kernel, mesh=mesh,
        out_type=jax.ShapeDtypeStruct((BHS, 2 * DH), jnp.float32),
        scratch_types=[
            pltpu.VMEM((TB,), jnp.int32),
            pltpu.VMEM((TB, 2 * DH), jnp.float32),
            pltpu.SemaphoreType.DMA,
        ],
    )
    def sc_gather(src_hbm, idx_hbm, out_hbm, idx_v, buf, sem):
        wid = lax.axis_index("s") * 2 + lax.axis_index("c")
        base = wid * RPW

        def body(j, carry):
            pltpu.sync_copy(idx_hbm.at[pl.ds(base + j * TB, TB)], idx_v)
            pltpu.async_copy(src_hbm.at[idx_v], buf, sem).wait()
            pltpu.sync_copy(buf, out_hbm.at[pl.ds(base + j * TB, TB)])
            return carry

        lax.fori_loop(0, NTB, body, 0)

    return sc_scatter, sc_gather


RB = 512
CPB = RB // CH
NBANDS = S // RB


NBPG = 8


def _attn_body(main_ref, prev_ref, out_ref):
    blk = main_ref[0, 0]
    prv = prev_ref[0, 0]
    iq = lax.broadcasted_iota(jnp.int32, (RB, RB + CH), 0) // CH
    jk = lax.broadcasted_iota(jnp.int32, (RB, RB + CH), 1) // CH
    band_ok = (jk == iq) | (jk == iq + 1)
    bandadd = jnp.where(band_ok, 0.0, -1e9)

    def do_band(rows, prev_rows, base):
        q = rows[:, 0:DH]
        kcat = jnp.concatenate([prev_rows[:, 0:DH], q], axis=0)
        vcat = jnp.concatenate([prev_rows[:, DH:2 * DH], rows[:, DH:2 * DH]],
                               axis=0)
        pcat = jnp.concatenate([prev_rows[:, 2 * DH:2 * DH + 1],
                                rows[:, 2 * DH:2 * DH + 1]], axis=0)
        nrm = jnp.sqrt(jnp.sum(kcat * kcat, -1, keepdims=True))
        kn = kcat / (nrm + 1e-6)
        qs = q * 0.125
        dots = lax.dot_general(qs, kn, (((1,), (1,)), ((), ())))
        kp = jnp.transpose(pcat)
        qp = rows[:, 2 * DH:2 * DH + 1]
        dots = jnp.where(kp > qp, -1e9, dots) + bandadd
        m = jnp.max(dots, -1, keepdims=True)
        e = jnp.exp(dots - m)
        attn = e / jnp.sum(e, -1, keepdims=True)
        o = jnp.dot(attn, vcat)
        out_ref[0, 0, base:base + RB, :] = jnp.concatenate([o, o], axis=1)

    do_band(blk[0:RB], prv, 0)
    for bb in range(1, NBPG):
        do_band(blk[bb * RB:(bb + 1) * RB],
                blk[bb * RB - CH:bb * RB], bb * RB)


def _run_attn(sorted_tbl):
    return pl.pallas_call(
        _attn_body,
        grid=(B, H, NBANDS // NBPG),
        in_specs=[
            pl.BlockSpec((1, 1, NBPG * RB, ROWW),
                         lambda b, h, i: (b, h, i, 0)),
            pl.BlockSpec((1, 1, CH, ROWW),
                         lambda b, h, i: (b, h,
                                          (i * CPB * NBPG - 1) % (S // CH),
                                          0)),
        ],
        out_specs=pl.BlockSpec((1, 1, NBPG * RB, 2 * DH),
                               lambda b, h, i: (b, h, i, 0)),
        out_shape=jax.ShapeDtypeStruct((B, H, S, 2 * DH), jnp.float32),
    )(sorted_tbl, sorted_tbl)


def _kv_body(keys_ref, Wk_ref, Wv2_ref, k2_ref, v2_ref):
    k = keys_ref[0]
    k2_ref[0] = jnp.dot(k, Wk_ref[...])
    v2_ref[0] = jnp.dot(k, Wv2_ref[...])


def _run_kv(keys, Wk, Wv2):
    return pl.pallas_call(
        _kv_body,
        grid=(B,),
        in_specs=[
            pl.BlockSpec((1, KS, D), lambda b: (b, 0, 0)),
            pl.BlockSpec((D, D), lambda b: (0, 0)),
            pl.BlockSpec((D, D), lambda b: (0, 0)),
        ],
        out_specs=[
            pl.BlockSpec((1, KS, D), lambda b: (b, 0, 0)),
            pl.BlockSpec((1, KS, D), lambda b: (b, 0, 0)),
        ],
        out_shape=[
            jax.ShapeDtypeStruct((B, KS, D), jnp.float32),
            jax.ShapeDtypeStruct((B, KS, D), jnp.float32),
        ],
    )(keys, Wk, Wv2)


def _tail_body(h0_ref, ou_ref, Wo_ref, g2_ref, be2_ref, Wq_ref,
               k2_ref, v2_ref, Wo2_ref, g3_ref, be3_ref,
               Wf1_ref, bf1_ref, Wf2_ref, bf2_ref,
               Wm_ref, bm_ref, ws_ref, bs_ref,
               mel_ref, stop_ref):
    acc = jnp.zeros((BS, D), jnp.float32)
    for hh in range(H):
        sl = slice(hh * DH, (hh + 1) * DH)
        acc = acc + jnp.dot(ou_ref[0, hh][:, 0:DH], Wo_ref[sl, :])
    h1 = h0_ref[0] + acc
    xn2 = _ln(h1, g2_ref[...], be2_ref[...])
    cross = jnp.zeros((BS, D), jnp.float32)
    for hh in range(H):
        sl = slice(hh * DH, (hh + 1) * DH)
        qh = jnp.dot(xn2, Wq_ref[:, sl])
        kh = k2_ref[0][:, sl]
        dh = lax.dot_general(qh, kh, (((1,), (1,)), ((), ()))) / 8.0
        m = jnp.max(dh, -1, keepdims=True)
        e = jnp.exp(dh - m)
        ah = e / jnp.sum(e, -1, keepdims=True)
        oh = jnp.dot(ah, v2_ref[0][:, sl])
        cross = cross + jnp.dot(oh, Wo2_ref[sl, :])
    h2 = h1 + cross
    xn3 = _ln(h2, g3_ref[...], be3_ref[...])
    f = jnp.maximum(jnp.dot(xn3, Wf1_ref[...]) + bf1_ref[...], 0.0)
    h3 = h2 + jnp.dot(f, Wf2_ref[...]) + bf2_ref[...]
    mel_ref[0] = jnp.dot(h3, Wm_ref[...]) + bm_ref[...]
    stop_ref[0] = (jnp.sum(h3 * ws_ref[...], -1, keepdims=True)
                   + bs_ref[0, 0])


def _run_tail(h0, o_u, Wo, g2, be2, Wq, k2, v2, Wo2, g3, be3,
              Wf1, bf1, Wf2, bf2, Wm, bm, Ws, bs):
    full = lambda shp: pl.BlockSpec(shp, lambda b, s: (0,) * len(shp))
    return pl.pallas_call(
        _tail_body,
        grid=(B, NSB),
        in_specs=[
            pl.BlockSpec((1, BS, D), lambda b, s: (b, s, 0)),
            pl.BlockSpec((1, H, BS, 2 * DH), lambda b, s: (b, 0, s, 0)),
            full((D, D)), full((1, D)), full((1, D)), full((D, D)),
            pl.BlockSpec((1, KS, D), lambda b, s: (b, 0, 0)),
            pl.BlockSpec((1, KS, D), lambda b, s: (b, 0, 0)),
            full((D, D)), full((1, D)), full((1, D)),
            full((D, DFF)), full((1, DFF)), full((DFF, D)), full((1, D)),
            full((D, NMEL)), full((1, NMEL)), full((1, D)), full((1, 1)),
        ],
        out_specs=[
            pl.BlockSpec((1, BS, NMEL), lambda b, s: (b, s, 0)),
            pl.BlockSpec((1, BS, 1), lambda b, s: (b, s, 0)),
        ],
        out_shape=[
            jax.ShapeDtypeStruct((B, S, NMEL), jnp.float32),
            jax.ShapeDtypeStruct((B, S, 1), jnp.float32),
        ],
    )(h0, o_u, Wo, g2.reshape(1, D), be2.reshape(1, D), Wq, k2, v2, Wo2,
      g3.reshape(1, D), be3.reshape(1, D), Wf1, bf1.reshape(1, DFF), Wf2,
      bf2.reshape(1, D), Wm, bm.reshape(1, NMEL), Ws.reshape(1, D),
      bs.reshape(1, 1))


@jax.jit
def kernel(input_, keys, pW1, pb1, pW2, pb2, alpha, rot, g1, be1, Wqk, Wv,
           Wo, g2, be2, Wq, Wk, Wv2, Wo2, g3, be3, Wf1, bf1, Wf2, bf2,
           Wm, bm, Ws, bs):
    pe_scaled = alpha * jnp.asarray(_PE_NP)
    h0, table, bkt_f = _run_prep(input_, pe_scaled, pW1, pb1, pW2, pb2,
                                 g1, be1, Wqk, Wv, rot)
    gdest = _run_dest(bkt_f)
    gidx = gdest.reshape(BHS)
    sc_scatter, sc_gather = _get_sc_kernels()
    sorted_tbl = sc_scatter(table.reshape(BHS, ROWW), gidx)
    o_sorted = _run_attn(sorted_tbl.reshape(B, H, S, ROWW))
    o_u = sc_gather(o_sorted.reshape(BHS, 2 * DH), gidx).reshape(
        B, H, S, 2 * DH)
    k2, v2 = _run_kv(keys, Wk, Wv2)
    mel, stop = _run_tail(h0, o_u, Wo, g2, be2, Wq, k2, v2, Wo2, g3, be3,
                          Wf1, bf1, Wf2, bf2, Wm, bm, Ws, bs)
    return (mel, stop)


# speedup vs baseline: 1.5054x; 1.0199x over previous
import functools
import numpy as np
import jax
import jax.numpy as jnp
from jax import lax
from jax.experimental import pallas as pl
from jax.experimental.pallas import tpu as pltpu
from jax.experimental.pallas import tpu_sc as plsc

B, S, D, H, NMEL = 2, 8192, 768, 12, 80
DH = D // H
NB = 128
CH = 64
KS = 512
DFF = 3072
PRE = 256
BH = B * H
BHS = BH * S
ROWW = 4 * DH
BS = 512
NSB = S // BS
RC = 128
Nb_CS = S // RC
NW = 32
RPW = BHS // NW
TB = 128
NTB = RPW // TB


def _sinusoid_np():
    pos = np.arange(S)[:, None].astype(np.float64)
    i = np.arange(D)[None, :]
    ang = pos / np.power(10000.0, (2 * (i // 2)) / float(D))
    return np.where(i % 2 == 0, np.sin(ang), np.cos(ang)).astype(np.float32)


_PE_NP = _sinusoid_np()


def _ln(x, g, b):
    m = jnp.mean(x, -1, keepdims=True)
    v = jnp.mean((x - m) * (x - m), -1, keepdims=True)
    return (x - m) / jnp.sqrt(v + 1e-5) * g + b


def _prep_body(inp_ref, pe_ref, pW1_ref, pb1_ref, pW2_ref, pb2_ref,
               g1_ref, be1_ref, Wqk_ref, Wv_ref, rot_ref,
               h0_ref, tab_ref, bkt_ref):
    x = inp_ref[0]
    h = jnp.maximum(jnp.dot(x, pW1_ref[...]) + pb1_ref[...], 0.0)
    h = jnp.maximum(jnp.dot(h, pW2_ref[...]) + pb2_ref[...], 0.0)
    h = h + pe_ref[...]
    h0_ref[0] = h
    xn = _ln(h, g1_ref[...], be1_ref[...])
    qk = jnp.dot(xn, Wqk_ref[...])
    v = jnp.dot(xn, Wv_ref[...])
    sblk = pl.program_id(1)
    posf = (jnp.float32(sblk * BS)
            + lax.broadcasted_iota(jnp.int32, (BS, DH), 0).astype(jnp.float32))
    io64 = lax.broadcasted_iota(jnp.int32, (BS, DH), 1).astype(jnp.float32)
    bkts = []
    for hh in range(H):
        sl = slice(hh * DH, (hh + 1) * DH)
        qkh = qk[:, sl]
        tab_ref[0, hh, :, 0:DH] = qkh
        tab_ref[0, hh, :, DH:2 * DH] = v[:, sl]
        tab_ref[0, hh, :, 2 * DH:3 * DH] = posf
        r = jnp.dot(qkh, rot_ref[hh])
        m1 = jnp.max(r, -1, keepdims=True)
        i1 = jnp.min(jnp.where(r == m1, io64, 1e9), -1)
        m2 = jnp.max(-r, -1, keepdims=True)
        i2 = jnp.min(jnp.where(-r == m2, io64, 1e9), -1)
        b = jnp.where(m1[:, 0] >= m2[:, 0], i1, 64.0 + i2)
        bkts.append(b)
    bkt_ref[0] = jnp.stack(bkts, axis=0)


def _run_prep(input_, pe_scaled, pW1, pb1, pW2, pb2, g1, be1, Wqk, Wv, rot):
    full = lambda shp: pl.BlockSpec(shp, lambda b, s: (0,) * len(shp))
    return pl.pallas_call(
        _prep_body,
        grid=(B, NSB),
        in_specs=[
            pl.BlockSpec((1, BS, NMEL), lambda b, s: (b, s, 0)),
            pl.BlockSpec((BS, D), lambda b, s: (s, 0)),
            full((NMEL, PRE)), full((1, PRE)), full((PRE, D)), full((1, D)),
            full((1, D)), full((1, D)), full((D, D)), full((D, D)),
            full((H, DH, NB // 2)),
        ],
        out_specs=[
            pl.BlockSpec((1, BS, D), lambda b, s: (b, s, 0)),
            pl.BlockSpec((1, H, BS, ROWW), lambda b, s: (b, 0, s, 0)),
            pl.BlockSpec((1, H, BS), lambda b, s: (b, 0, s)),
        ],
        out_shape=[
            jax.ShapeDtypeStruct((B, S, D), jnp.float32),
            jax.ShapeDtypeStruct((B, H, S, ROWW), jnp.float32),
            jax.ShapeDtypeStruct((B, H, S), jnp.float32),
        ],
    )(input_, pe_scaled, pW1, pb1.reshape(1, PRE), pW2, pb2.reshape(1, D),
      g1.reshape(1, D), be1.reshape(1, D), Wqk, Wv, rot)


def _dest_body(bk_ref, out_ref):
    bk = bk_ref[0]
    iob = lax.broadcasted_iota(jnp.int32, (1, NB), 1).astype(jnp.float32)
    oh3 = (bk[:, :, None] == iob[None]).astype(jnp.float32)
    cnt = jnp.sum(oh3, axis=1)
    eq = (bk[:, :, None] == bk[:, None, :]).astype(jnp.float32)
    tri = (lax.broadcasted_iota(jnp.int32, (RC, RC), 1)
           < lax.broadcasted_iota(jnp.int32, (RC, RC), 0)).astype(jnp.float32)
    rank_local = jnp.sum(eq * tri[None], axis=2)
    ioc = lax.broadcasted_iota(jnp.int32, (Nb_CS, Nb_CS), 0)
    ioc2 = lax.broadcasted_iota(jnp.int32, (Nb_CS, Nb_CS), 1)
    ltri = (ioc2 < ioc).astype(jnp.float32)
    cp = jnp.sum(ltri[:, :, None] * cnt[None, :, :], axis=1)
    total = jnp.sum(cnt, axis=0, keepdims=True)
    totc = jnp.transpose(total)
    iom = lax.broadcasted_iota(jnp.int32, (NB, NB), 0)
    ion = lax.broadcasted_iota(jnp.int32, (NB, NB), 1)
    bstart = jnp.sum(jnp.where(iom < ion, totc, 0.0), axis=0,
                     keepdims=True)
    base = bstart + cp
    term1 = jnp.sum(oh3 * base[:, None, :], axis=2)
    dest = term1 + rank_local
    gdest = dest.astype(jnp.int32) + pl.program_id(0) * S
    out_ref[0] = gdest


def _run_dest(bkt_f):
    return pl.pallas_call(
        _dest_body,
        grid=(BH,),
        in_specs=[pl.BlockSpec((1, Nb_CS, RC), lambda i: (i, 0, 0))],
        out_specs=pl.BlockSpec((1, Nb_CS, RC), lambda i: (i, 0, 0)),
        out_shape=jax.ShapeDtypeStruct((BH, Nb_CS, RC), jnp.int32),
    )(bkt_f.reshape(BH, Nb_CS, RC))


@functools.lru_cache(maxsize=1)
def _get_sc_kernels():
    mesh = plsc.VectorSubcoreMesh(core_axis_name="c", subcore_axis_name="s")

    @functools.partial(
        pl.kernel, mesh=mesh,
        out_type=jax.ShapeDtypeStruct((BHS, ROWW), jnp.float32),
        scratch_types=[
            pltpu.VMEM((TB,), jnp.int32),
            pltpu.VMEM((TB, ROWW), jnp.float32),
            pltpu.SemaphoreType.DMA,
        ],
    )
    def sc_scatter(src_hbm, idx_hbm, out_hbm, idx_v, buf, sem):
        wid = lax.axis_index("s") * 2 + lax.axis_index("c")
        base = wid * RPW

        def body(j, carry):
            pltpu.sync_copy(idx_hbm.at[pl.ds(base + j * TB, TB)], idx_v)
            pltpu.sync_copy(src_hbm.at[pl.ds(base + j * TB, TB)], buf)
            pltpu.async_copy(buf, out_hbm.at[idx_v], sem).wait()
            return carry

        lax.fori_loop(0, NTB, body, 0)

    @functools.partial(
        pl.kernel, mesh=mesh,
        out_type=jax.ShapeDtypeStruct((BHS, 2 * DH), jnp.float32),
        scratch_types=[
            pltpu.VMEM((TB,), jnp.int32),
            pltpu.VMEM((TB, 2 * DH), jnp.float32),
            pltpu.SemaphoreType.DMA,
        ],
    )
    def sc_gather(src_hbm, idx_hbm, out_hbm, idx_v, buf, sem):
        wid = lax.axis_index("s") * 2 + lax.axis_index("c")
        base = wid * RPW

        def body(j, carry):
            pltpu.sync_copy(idx_hbm.at[pl.ds(base + j * TB, TB)], idx_v)
            pltpu.async_copy(src_hbm.at[idx_v], buf, sem).wait()
            pltpu.sync_copy(buf, out_hbm.at[pl.ds(base + j * TB, TB)])
            return carry

        lax.fori_loop(0, NTB, body, 0)

    return sc_scatter, sc_gather


RB = 512
CPB = RB // CH
NBANDS = S // RB


NBPG = 16


def _attn_body(main_ref, prev_ref, out_ref):
    blk = main_ref[0, 0]
    prv = prev_ref[0, 0]
    iq = lax.broadcasted_iota(jnp.int32, (RB, RB + CH), 0) // CH
    jk = lax.broadcasted_iota(jnp.int32, (RB, RB + CH), 1) // CH
    band_ok = (jk == iq) | (jk == iq + 1)
    bandadd = jnp.where(band_ok, 0.0, -1e9)

    def do_band(rows, prev_rows, base):
        q = rows[:, 0:DH]
        kcat = jnp.concatenate([prev_rows[:, 0:DH], q], axis=0)
        vcat = jnp.concatenate([prev_rows[:, DH:2 * DH], rows[:, DH:2 * DH]],
                               axis=0)
        pcat = jnp.concatenate([prev_rows[:, 2 * DH:2 * DH + 1],
                                rows[:, 2 * DH:2 * DH + 1]], axis=0)
        nrm = jnp.sqrt(jnp.sum(kcat * kcat, -1, keepdims=True))
        kn = kcat / (nrm + 1e-6)
        qs = q * 0.125
        dots = lax.dot_general(qs, kn, (((1,), (1,)), ((), ())))
        kp = jnp.transpose(pcat)
        qp = rows[:, 2 * DH:2 * DH + 1]
        dots = jnp.where(kp > qp, -1e9, dots) + bandadd
        m = jnp.max(dots, -1, keepdims=True)
        e = jnp.exp(dots - m)
        attn = e / jnp.sum(e, -1, keepdims=True)
        o = jnp.dot(attn, vcat)
        out_ref[0, 0, base:base + RB, :] = jnp.concatenate([o, o], axis=1)

    do_band(blk[0:RB], prv, 0)
    for bb in range(1, NBPG):
        do_band(blk[bb * RB:(bb + 1) * RB],
                blk[bb * RB - CH:bb * RB], bb * RB)


def _run_attn(sorted_tbl):
    return pl.pallas_call(
        _attn_body,
        grid=(B, H, NBANDS // NBPG),
        in_specs=[
            pl.BlockSpec((1, 1, NBPG * RB, ROWW),
                         lambda b, h, i: (b, h, i, 0)),
            pl.BlockSpec((1, 1, CH, ROWW),
                         lambda b, h, i: (b, h,
                                          (i * CPB * NBPG - 1) % (S // CH),
                                          0)),
        ],
        out_specs=pl.BlockSpec((1, 1, NBPG * RB, 2 * DH),
                               lambda b, h, i: (b, h, i, 0)),
        out_shape=jax.ShapeDtypeStruct((B, H, S, 2 * DH), jnp.float32),
    )(sorted_tbl, sorted_tbl)


def _kv_body(keys_ref, Wk_ref, Wv2_ref, k2_ref, v2_ref):
    k = keys_ref[0]
    k2_ref[0] = jnp.dot(k, Wk_ref[...])
    v2_ref[0] = jnp.dot(k, Wv2_ref[...])


def _run_kv(keys, Wk, Wv2):
    return pl.pallas_call(
        _kv_body,
        grid=(B,),
        in_specs=[
            pl.BlockSpec((1, KS, D), lambda b: (b, 0, 0)),
            pl.BlockSpec((D, D), lambda b: (0, 0)),
            pl.BlockSpec((D, D), lambda b: (0, 0)),
        ],
        out_specs=[
            pl.BlockSpec((1, KS, D), lambda b: (b, 0, 0)),
            pl.BlockSpec((1, KS, D), lambda b: (b, 0, 0)),
        ],
        out_shape=[
            jax.ShapeDtypeStruct((B, KS, D), jnp.float32),
            jax.ShapeDtypeStruct((B, KS, D), jnp.float32),
        ],
    )(keys, Wk, Wv2)


def _tail_body(h0_ref, ou_ref, Wo_ref, g2_ref, be2_ref, Wq_ref,
               k2_ref, v2_ref, Wo2_ref, g3_ref, be3_ref,
               Wf1_ref, bf1_ref, Wf2_ref, bf2_ref,
               Wm_ref, bm_ref, ws_ref, bs_ref,
               mel_ref, stop_ref):
    acc = jnp.zeros((BS, D), jnp.float32)
    for hh in range(H):
        sl = slice(hh * DH, (hh + 1) * DH)
        acc = acc + jnp.dot(ou_ref[0, hh][:, 0:DH], Wo_ref[sl, :])
    h1 = h0_ref[0] + acc
    xn2 = _ln(h1, g2_ref[...], be2_ref[...])
    cross = jnp.zeros((BS, D), jnp.float32)
    for hh in range(H):
        sl = slice(hh * DH, (hh + 1) * DH)
        qh = jnp.dot(xn2, Wq_ref[:, sl])
        kh = k2_ref[0][:, sl]
        dh = lax.dot_general(qh, kh, (((1,), (1,)), ((), ()))) / 8.0
        m = jnp.max(dh, -1, keepdims=True)
        e = jnp.exp(dh - m)
        ah = e / jnp.sum(e, -1, keepdims=True)
        oh = jnp.dot(ah, v2_ref[0][:, sl])
        cross = cross + jnp.dot(oh, Wo2_ref[sl, :])
    h2 = h1 + cross
    xn3 = _ln(h2, g3_ref[...], be3_ref[...])
    f = jnp.maximum(jnp.dot(xn3, Wf1_ref[...]) + bf1_ref[...], 0.0)
    h3 = h2 + jnp.dot(f, Wf2_ref[...]) + bf2_ref[...]
    mel_ref[0] = jnp.dot(h3, Wm_ref[...]) + bm_ref[...]
    stop_ref[0] = (jnp.sum(h3 * ws_ref[...], -1, keepdims=True)
                   + bs_ref[0, 0])


def _run_tail(h0, o_u, Wo, g2, be2, Wq, k2, v2, Wo2, g3, be3,
              Wf1, bf1, Wf2, bf2, Wm, bm, Ws, bs):
    full = lambda shp: pl.BlockSpec(shp, lambda b, s: (0,) * len(shp))
    return pl.pallas_call(
        _tail_body,
        grid=(B, NSB),
        in_specs=[
            pl.BlockSpec((1, BS, D), lambda b, s: (b, s, 0)),
            pl.BlockSpec((1, H, BS, 2 * DH), lambda b, s: (b, 0, s, 0)),
            full((D, D)), full((1, D)), full((1, D)), full((D, D)),
            pl.BlockSpec((1, KS, D), lambda b, s: (b, 0, 0)),
            pl.BlockSpec((1, KS, D), lambda b, s: (b, 0, 0)),
            full((D, D)), full((1, D)), full((1, D)),
            full((D, DFF)), full((1, DFF)), full((DFF, D)), full((1, D)),
            full((D, NMEL)), full((1, NMEL)), full((1, D)), full((1, 1)),
        ],
        out_specs=[
            pl.BlockSpec((1, BS, NMEL), lambda b, s: (b, s, 0)),
            pl.BlockSpec((1, BS, 1), lambda b, s: (b, s, 0)),
        ],
        out_shape=[
            jax.ShapeDtypeStruct((B, S, NMEL), jnp.float32),
            jax.ShapeDtypeStruct((B, S, 1), jnp.float32),
        ],
    )(h0, o_u, Wo, g2.reshape(1, D), be2.reshape(1, D), Wq, k2, v2, Wo2,
      g3.reshape(1, D), be3.reshape(1, D), Wf1, bf1.reshape(1, DFF), Wf2,
      bf2.reshape(1, D), Wm, bm.reshape(1, NMEL), Ws.reshape(1, D),
      bs.reshape(1, 1))


@jax.jit
def kernel(input_, keys, pW1, pb1, pW2, pb2, alpha, rot, g1, be1, Wqk, Wv,
           Wo, g2, be2, Wq, Wk, Wv2, Wo2, g3, be3, Wf1, bf1, Wf2, bf2,
           Wm, bm, Ws, bs):
    pe_scaled = alpha * jnp.asarray(_PE_NP)
    h0, table, bkt_f = _run_prep(input_, pe_scaled, pW1, pb1, pW2, pb2,
                                 g1, be1, Wqk, Wv, rot)
    gdest = _run_dest(bkt_f)
    gidx = gdest.reshape(BHS)
    sc_scatter, sc_gather = _get_sc_kernels()
    sorted_tbl = sc_scatter(table.reshape(BHS, ROWW), gidx)
    o_sorted = _run_attn(sorted_tbl.reshape(B, H, S, ROWW))
    o_u = sc_gather(o_sorted.reshape(BHS, 2 * DH), gidx).reshape(
        B, H, S, 2 * DH)
    k2, v2 = _run_kv(keys, Wk, Wv2)
    mel, stop = _run_tail(h0, o_u, Wo, g2, be2, Wq, k2, v2, Wo2, g3, be3,
                          Wf1, bf1, Wf2, bf2, Wm, bm, Ws, bs)
    return (mel, stop)


# speedup vs baseline: 1.5695x; 1.0425x over previous
import functools
import numpy as np
import jax
import jax.numpy as jnp
from jax import lax
from jax.experimental import pallas as pl
from jax.experimental.pallas import tpu as pltpu
from jax.experimental.pallas import tpu_sc as plsc

B, S, D, H, NMEL = 2, 8192, 768, 12, 80
DH = D // H
NB = 128
CH = 64
KS = 512
DFF = 3072
PRE = 256
BH = B * H
BHS = BH * S
ROWW = 4 * DH
BS = 512
NSB = S // BS
RC = 128
Nb_CS = S // RC
NW = 32
RPW = BHS // NW
TB = 128
NTB = RPW // TB


def _sinusoid_np():
    pos = np.arange(S)[:, None].astype(np.float64)
    i = np.arange(D)[None, :]
    ang = pos / np.power(10000.0, (2 * (i // 2)) / float(D))
    return np.where(i % 2 == 0, np.sin(ang), np.cos(ang)).astype(np.float32)


_PE_NP = _sinusoid_np()


def _ln(x, g, b):
    m = jnp.mean(x, -1, keepdims=True)
    v = jnp.mean((x - m) * (x - m), -1, keepdims=True)
    return (x - m) / jnp.sqrt(v + 1e-5) * g + b


def _prep_body(inp_ref, pe_ref, pW1_ref, pb1_ref, pW2_ref, pb2_ref,
               g1_ref, be1_ref, Wqk_ref, Wv_ref, rot_ref,
               h0_ref, tab_ref, bkt_ref):
    x = inp_ref[0]
    h = jnp.maximum(jnp.dot(x, pW1_ref[...]) + pb1_ref[...], 0.0)
    h = jnp.maximum(jnp.dot(h, pW2_ref[...]) + pb2_ref[...], 0.0)
    h = h + pe_ref[...]
    h0_ref[0] = h
    xn = _ln(h, g1_ref[...], be1_ref[...])
    qk = jnp.dot(xn, Wqk_ref[...])
    v = jnp.dot(xn, Wv_ref[...])
    sblk = pl.program_id(1)
    posf = (jnp.float32(sblk * BS)
            + lax.broadcasted_iota(jnp.int32, (BS, DH), 0).astype(jnp.float32))
    io64 = lax.broadcasted_iota(jnp.int32, (BS, DH), 1).astype(jnp.float32)
    bkts = []
    for hh in range(H):
        sl = slice(hh * DH, (hh + 1) * DH)
        qkh = qk[:, sl]
        tab_ref[0, hh, :, 0:DH] = qkh
        tab_ref[0, hh, :, DH:2 * DH] = v[:, sl]
        tab_ref[0, hh, :, 2 * DH:3 * DH] = posf
        r = jnp.dot(qkh, rot_ref[hh])
        m1 = jnp.max(r, -1, keepdims=True)
        i1 = jnp.min(jnp.where(r == m1, io64, 1e9), -1)
        m2 = jnp.max(-r, -1, keepdims=True)
        i2 = jnp.min(jnp.where(-r == m2, io64, 1e9), -1)
        b = jnp.where(m1[:, 0] >= m2[:, 0], i1, 64.0 + i2)
        bkts.append(b)
    bkt_ref[0] = jnp.stack(bkts, axis=0)


def _run_prep(input_, pe_scaled, pW1, pb1, pW2, pb2, g1, be1, Wqk, Wv, rot):
    full = lambda shp: pl.BlockSpec(shp, lambda b, s: (0,) * len(shp))
    return pl.pallas_call(
        _prep_body,
        grid=(B, NSB),
        in_specs=[
            pl.BlockSpec((1, BS, NMEL), lambda b, s: (b, s, 0)),
            pl.BlockSpec((BS, D), lambda b, s: (s, 0)),
            full((NMEL, PRE)), full((1, PRE)), full((PRE, D)), full((1, D)),
            full((1, D)), full((1, D)), full((D, D)), full((D, D)),
            full((H, DH, NB // 2)),
        ],
        out_specs=[
            pl.BlockSpec((1, BS, D), lambda b, s: (b, s, 0)),
            pl.BlockSpec((1, H, BS, ROWW), lambda b, s: (b, 0, s, 0)),
            pl.BlockSpec((1, H, BS), lambda b, s: (b, 0, s)),
        ],
        out_shape=[
            jax.ShapeDtypeStruct((B, S, D), jnp.float32),
            jax.ShapeDtypeStruct((B, H, S, ROWW), jnp.float32),
            jax.ShapeDtypeStruct((B, H, S), jnp.float32),
        ],
    )(input_, pe_scaled, pW1, pb1.reshape(1, PRE), pW2, pb2.reshape(1, D),
      g1.reshape(1, D), be1.reshape(1, D), Wqk, Wv, rot)


def _dest_body(bk_ref, out_ref):
    bk = bk_ref[0]
    iob = lax.broadcasted_iota(jnp.int32, (1, NB), 1).astype(jnp.float32)
    oh3 = (bk[:, :, None] == iob[None]).astype(jnp.float32)
    cnt = jnp.sum(oh3, axis=1)
    eq = (bk[:, :, None] == bk[:, None, :]).astype(jnp.float32)
    tri = (lax.broadcasted_iota(jnp.int32, (RC, RC), 1)
           < lax.broadcasted_iota(jnp.int32, (RC, RC), 0)).astype(jnp.float32)
    rank_local = jnp.sum(eq * tri[None], axis=2)
    ioc = lax.broadcasted_iota(jnp.int32, (Nb_CS, Nb_CS), 0)
    ioc2 = lax.broadcasted_iota(jnp.int32, (Nb_CS, Nb_CS), 1)
    ltri = (ioc2 < ioc).astype(jnp.float32)
    cp = jnp.sum(ltri[:, :, None] * cnt[None, :, :], axis=1)
    total = jnp.sum(cnt, axis=0, keepdims=True)
    totc = jnp.transpose(total)
    iom = lax.broadcasted_iota(jnp.int32, (NB, NB), 0)
    ion = lax.broadcasted_iota(jnp.int32, (NB, NB), 1)
    bstart = jnp.sum(jnp.where(iom < ion, totc, 0.0), axis=0,
                     keepdims=True)
    base = bstart + cp
    term1 = jnp.sum(oh3 * base[:, None, :], axis=2)
    dest = term1 + rank_local
    gdest = dest.astype(jnp.int32) + pl.program_id(0) * S
    out_ref[0] = gdest


def _run_dest(bkt_f):
    return pl.pallas_call(
        _dest_body,
        grid=(BH,),
        in_specs=[pl.BlockSpec((1, Nb_CS, RC), lambda i: (i, 0, 0))],
        out_specs=pl.BlockSpec((1, Nb_CS, RC), lambda i: (i, 0, 0)),
        out_shape=jax.ShapeDtypeStruct((BH, Nb_CS, RC), jnp.int32),
    )(bkt_f.reshape(BH, Nb_CS, RC))


@functools.lru_cache(maxsize=1)
def _get_sc_kernels():
    mesh = plsc.VectorSubcoreMesh(core_axis_name="c", subcore_axis_name="s")

    @functools.partial(
        pl.kernel, mesh=mesh,
        out_type=jax.ShapeDtypeStruct((BHS, ROWW), jnp.float32),
        scratch_types=[
            pltpu.VMEM((TB,), jnp.int32),
            pltpu.VMEM((TB,), jnp.int32),
            pltpu.VMEM((TB, ROWW), jnp.float32),
            pltpu.VMEM((TB, ROWW), jnp.float32),
            pltpu.SemaphoreType.DMA,
            pltpu.SemaphoreType.DMA,
            pltpu.SemaphoreType.DMA,
            pltpu.SemaphoreType.DMA,
        ],
    )
    def sc_scatter(src_hbm, idx_hbm, out_hbm, idx_a, idx_b, buf_a, buf_b,
                   sem_la, sem_lb, sem_sa, sem_sb):
        wid = lax.axis_index("s") * 2 + lax.axis_index("c")
        base = wid * RPW

        def body(p, carry):
            j0 = base + (2 * p) * TB
            j1 = base + (2 * p + 1) * TB
            lia = pltpu.async_copy(idx_hbm.at[pl.ds(j0, TB)], idx_a, sem_la)
            lda = pltpu.async_copy(src_hbm.at[pl.ds(j0, TB)], buf_a, sem_la)
            lib = pltpu.async_copy(idx_hbm.at[pl.ds(j1, TB)], idx_b, sem_lb)
            ldb = pltpu.async_copy(src_hbm.at[pl.ds(j1, TB)], buf_b, sem_lb)
            lia.wait()
            lda.wait()
            sa = pltpu.async_copy(buf_a, out_hbm.at[idx_a], sem_sa)
            lib.wait()
            ldb.wait()
            sb = pltpu.async_copy(buf_b, out_hbm.at[idx_b], sem_sb)
            sa.wait()
            sb.wait()
            return carry

        lax.fori_loop(0, NTB // 2, body, 0)

    @functools.partial(
        pl.kernel, mesh=mesh,
        out_type=jax.ShapeDtypeStruct((BHS, 2 * DH), jnp.float32),
        scratch_types=[
            pltpu.VMEM((TB,), jnp.int32),
            pltpu.VMEM((TB,), jnp.int32),
            pltpu.VMEM((TB, 2 * DH), jnp.float32),
            pltpu.VMEM((TB, 2 * DH), jnp.float32),
            pltpu.SemaphoreType.DMA,
            pltpu.SemaphoreType.DMA,
            pltpu.SemaphoreType.DMA,
            pltpu.SemaphoreType.DMA,
        ],
    )
    def sc_gather(src_hbm, idx_hbm, out_hbm, idx_a, idx_b, buf_a, buf_b,
                  sem_la, sem_lb, sem_sa, sem_sb):
        wid = lax.axis_index("s") * 2 + lax.axis_index("c")
        base = wid * RPW

        def body(p, carry):
            j0 = base + (2 * p) * TB
            j1 = base + (2 * p + 1) * TB
            lia = pltpu.async_copy(idx_hbm.at[pl.ds(j0, TB)], idx_a, sem_la)
            lib = pltpu.async_copy(idx_hbm.at[pl.ds(j1, TB)], idx_b, sem_lb)
            lia.wait()
            ga = pltpu.async_copy(src_hbm.at[idx_a], buf_a, sem_sa)
            lib.wait()
            gb = pltpu.async_copy(src_hbm.at[idx_b], buf_b, sem_sb)
            ga.wait()
            pltpu.sync_copy(buf_a, out_hbm.at[pl.ds(j0, TB)])
            gb.wait()
            pltpu.sync_copy(buf_b, out_hbm.at[pl.ds(j1, TB)])
            return carry

        lax.fori_loop(0, NTB // 2, body, 0)

    return sc_scatter, sc_gather


RB = 512
CPB = RB // CH
NBANDS = S // RB


NBPG = 16


def _attn_body(main_ref, prev_ref, out_ref):
    blk = main_ref[0, 0]
    prv = prev_ref[0, 0]
    iq = lax.broadcasted_iota(jnp.int32, (RB, RB + CH), 0) // CH
    jk = lax.broadcasted_iota(jnp.int32, (RB, RB + CH), 1) // CH
    band_ok = (jk == iq) | (jk == iq + 1)
    bandadd = jnp.where(band_ok, 0.0, -1e9)

    def do_band(rows, prev_rows, base):
        q = rows[:, 0:DH]
        kcat = jnp.concatenate([prev_rows[:, 0:DH], q], axis=0)
        vcat = jnp.concatenate([prev_rows[:, DH:2 * DH], rows[:, DH:2 * DH]],
                               axis=0)
        pcat = jnp.concatenate([prev_rows[:, 2 * DH:2 * DH + 1],
                                rows[:, 2 * DH:2 * DH + 1]], axis=0)
        nrm = jnp.sqrt(jnp.sum(kcat * kcat, -1, keepdims=True))
        kn = kcat / (nrm + 1e-6)
        qs = q * 0.125
        dots = lax.dot_general(qs, kn, (((1,), (1,)), ((), ())))
        kp = jnp.transpose(pcat)
        qp = rows[:, 2 * DH:2 * DH + 1]
        dots = jnp.where(kp > qp, -1e9, dots) + bandadd
        m = jnp.max(dots, -1, keepdims=True)
        e = jnp.exp(dots - m)
        attn = e / jnp.sum(e, -1, keepdims=True)
        o = jnp.dot(attn, vcat)
        out_ref[0, 0, base:base + RB, :] = jnp.concatenate([o, o], axis=1)

    do_band(blk[0:RB], prv, 0)
    for bb in range(1, NBPG):
        do_band(blk[bb * RB:(bb + 1) * RB],
                blk[bb * RB - CH:bb * RB], bb * RB)


def _run_attn(sorted_tbl):
    return pl.pallas_call(
        _attn_body,
        grid=(B, H, NBANDS // NBPG),
        in_specs=[
            pl.BlockSpec((1, 1, NBPG * RB, ROWW),
                         lambda b, h, i: (b, h, i, 0)),
            pl.BlockSpec((1, 1, CH, ROWW),
                         lambda b, h, i: (b, h,
                                          (i * CPB * NBPG - 1) % (S // CH),
                                          0)),
        ],
        out_specs=pl.BlockSpec((1, 1, NBPG * RB, 2 * DH),
                               lambda b, h, i: (b, h, i, 0)),
        out_shape=jax.ShapeDtypeStruct((B, H, S, 2 * DH), jnp.float32),
    )(sorted_tbl, sorted_tbl)


def _kv_body(keys_ref, Wk_ref, Wv2_ref, k2_ref, v2_ref):
    k = keys_ref[0]
    k2_ref[0] = jnp.dot(k, Wk_ref[...])
    v2_ref[0] = jnp.dot(k, Wv2_ref[...])


def _run_kv(keys, Wk, Wv2):
    return pl.pallas_call(
        _kv_body,
        grid=(B,),
        in_specs=[
            pl.BlockSpec((1, KS, D), lambda b: (b, 0, 0)),
            pl.BlockSpec((D, D), lambda b: (0, 0)),
            pl.BlockSpec((D, D), lambda b: (0, 0)),
        ],
        out_specs=[
            pl.BlockSpec((1, KS, D), lambda b: (b, 0, 0)),
            pl.BlockSpec((1, KS, D), lambda b: (b, 0, 0)),
        ],
        out_shape=[
            jax.ShapeDtypeStruct((B, KS, D), jnp.float32),
            jax.ShapeDtypeStruct((B, KS, D), jnp.float32),
        ],
    )(keys, Wk, Wv2)


def _tail_body(h0_ref, ou_ref, Wo_ref, g2_ref, be2_ref, Wq_ref,
               k2_ref, v2_ref, Wo2_ref, g3_ref, be3_ref,
               Wf1_ref, bf1_ref, Wf2_ref, bf2_ref,
               Wm_ref, bm_ref, ws_ref, bs_ref,
               mel_ref, stop_ref):
    acc = jnp.zeros((BS, D), jnp.float32)
    for hh in range(H):
        sl = slice(hh * DH, (hh + 1) * DH)
        acc = acc + jnp.dot(ou_ref[0, hh][:, 0:DH], Wo_ref[sl, :])
    h1 = h0_ref[0] + acc
    xn2 = _ln(h1, g2_ref[...], be2_ref[...])
    cross = jnp.zeros((BS, D), jnp.float32)
    for hh in range(H):
        sl = slice(hh * DH, (hh + 1) * DH)
        qh = jnp.dot(xn2, Wq_ref[:, sl])
        kh = k2_ref[0][:, sl]
        dh = lax.dot_general(qh, kh, (((1,), (1,)), ((), ()))) / 8.0
        m = jnp.max(dh, -1, keepdims=True)
        e = jnp.exp(dh - m)
        ah = e / jnp.sum(e, -1, keepdims=True)
        oh = jnp.dot(ah, v2_ref[0][:, sl])
        cross = cross + jnp.dot(oh, Wo2_ref[sl, :])
    h2 = h1 + cross
    xn3 = _ln(h2, g3_ref[...], be3_ref[...])
    f = jnp.maximum(jnp.dot(xn3, Wf1_ref[...]) + bf1_ref[...], 0.0)
    h3 = h2 + jnp.dot(f, Wf2_ref[...]) + bf2_ref[...]
    mel_ref[0] = jnp.dot(h3, Wm_ref[...]) + bm_ref[...]
    stop_ref[0] = (jnp.sum(h3 * ws_ref[...], -1, keepdims=True)
                   + bs_ref[0, 0])


def _run_tail(h0, o_u, Wo, g2, be2, Wq, k2, v2, Wo2, g3, be3,
              Wf1, bf1, Wf2, bf2, Wm, bm, Ws, bs):
    full = lambda shp: pl.BlockSpec(shp, lambda b, s: (0,) * len(shp))
    return pl.pallas_call(
        _tail_body,
        grid=(B, NSB),
        in_specs=[
            pl.BlockSpec((1, BS, D), lambda b, s: (b, s, 0)),
            pl.BlockSpec((1, H, BS, 2 * DH), lambda b, s: (b, 0, s, 0)),
            full((D, D)), full((1, D)), full((1, D)), full((D, D)),
            pl.BlockSpec((1, KS, D), lambda b, s: (b, 0, 0)),
            pl.BlockSpec((1, KS, D), lambda b, s: (b, 0, 0)),
            full((D, D)), full((1, D)), full((1, D)),
            full((D, DFF)), full((1, DFF)), full((DFF, D)), full((1, D)),
            full((D, NMEL)), full((1, NMEL)), full((1, D)), full((1, 1)),
        ],
        out_specs=[
            pl.BlockSpec((1, BS, NMEL), lambda b, s: (b, s, 0)),
            pl.BlockSpec((1, BS, 1), lambda b, s: (b, s, 0)),
        ],
        out_shape=[
            jax.ShapeDtypeStruct((B, S, NMEL), jnp.float32),
            jax.ShapeDtypeStruct((B, S, 1), jnp.float32),
        ],
    )(h0, o_u, Wo, g2.reshape(1, D), be2.reshape(1, D), Wq, k2, v2, Wo2,
      g3.reshape(1, D), be3.reshape(1, D), Wf1, bf1.reshape(1, DFF), Wf2,
      bf2.reshape(1, D), Wm, bm.reshape(1, NMEL), Ws.reshape(1, D),
      bs.reshape(1, 1))


@jax.jit
def kernel(input_, keys, pW1, pb1, pW2, pb2, alpha, rot, g1, be1, Wqk, Wv,
           Wo, g2, be2, Wq, Wk, Wv2, Wo2, g3, be3, Wf1, bf1, Wf2, bf2,
           Wm, bm, Ws, bs):
    pe_scaled = alpha * jnp.asarray(_PE_NP)
    h0, table, bkt_f = _run_prep(input_, pe_scaled, pW1, pb1, pW2, pb2,
                                 g1, be1, Wqk, Wv, rot)
    gdest = _run_dest(bkt_f)
    gidx = gdest.reshape(BHS)
    sc_scatter, sc_gather = _get_sc_kernels()
    sorted_tbl = sc_scatter(table.reshape(BHS, ROWW), gidx)
    o_sorted = _run_attn(sorted_tbl.reshape(B, H, S, ROWW))
    o_u = sc_gather(o_sorted.reshape(BHS, 2 * DH), gidx).reshape(
        B, H, S, 2 * DH)
    k2, v2 = _run_kv(keys, Wk, Wv2)
    mel, stop = _run_tail(h0, o_u, Wo, g2, be2, Wq, k2, v2, Wo2, g3, be3,
                          Wf1, bf1, Wf2, bf2, Wm, bm, Ws, bs)
    return (mel, stop)
